# Initial kernel scaffold; baseline (speedup 1.0000x reference)
import functools

import jax
import jax.numpy as jnp
from jax import lax
from jax.experimental import pallas as pl
from jax.experimental.pallas import tpu as pltpu
from jax.experimental.pallas import tpu_sc as plsc

N_NODES = 10000
N_EDGES = 160000
D_IN = 256
D_HID = 256
D_OUT = 40
D_OUT_PAD = 64

NS = 16
NC = 2
LB = 128
EPT = N_EDGES // NS
NB1 = (EPT + LB - 1) // LB
EPW = N_EDGES // (NS * NC)
NBD = (EPW + LB - 1) // LB
NPAD = N_NODES + 16
RPT = NPAD // NS

BLK = 500
GRID = N_NODES // BLK


def _mesh():
    return plsc.VectorSubcoreMesh(core_axis_name="c", subcore_axis_name="s")


def _deg_body(colp, zeros, ones_hbm, dega, degb, colv, ones, acc, sem):
    c = lax.axis_index("c")
    s = lax.axis_index("s")
    wid = s * NC + c
    pltpu.sync_copy(zeros.at[pl.ds(s * RPT, RPT)], acc.at[pl.ds(s * RPT, RPT)])
    pltpu.sync_copy(ones_hbm, ones)
    pltpu.sync_copy(colp.at[wid], colv)
    plsc.subcore_barrier()

    def body(b, carry):
        pltpu.sync_copy(ones, acc.at[colv.at[b]], add=True)
        return carry

    lax.fori_loop(0, NBD, body, 0)
    plsc.subcore_barrier()

    @pl.when(c == 0)
    def _():
        pltpu.sync_copy(acc.at[pl.ds(s * RPT, RPT)], dega.at[pl.ds(s * RPT, RPT)])

    @pl.when(c == 1)
    def _():
        pltpu.sync_copy(acc.at[pl.ds(s * RPT, RPT)], degb.at[pl.ds(s * RPT, RPT)])


def _make_deg_kernel():
    return pl.kernel(
        _deg_body,
        out_type=(
            jax.ShapeDtypeStruct((NPAD, 16), jnp.float32),
            jax.ShapeDtypeStruct((NPAD, 16), jnp.float32),
        ),
        mesh=_mesh(),
        scratch_types=[
            pltpu.VMEM((NBD, LB), jnp.int32),
            pltpu.VMEM((LB, 16), jnp.float32),
            pltpu.VMEM_SHARED((NPAD, 16), jnp.float32),
            pltpu.SemaphoreType.DMA,
        ],
    )


def _agg_body(hs_a, hs_b, rowp, colp, zeros, out_a, out_b,
              rowv, colv, buf0, buf1, acc, sem):
    c = lax.axis_index("c")
    s = lax.axis_index("s")
    pltpu.sync_copy(zeros.at[pl.ds(s * RPT, RPT)], acc.at[pl.ds(s * RPT, RPT)])
    pltpu.sync_copy(rowp.at[s], rowv)
    pltpu.sync_copy(colp.at[s], colv)
    plsc.subcore_barrier()

    def edge_loop(hs):
        pltpu.async_copy(hs.at[rowv.at[0]], buf0, sem)

        def body(g, carry):
            b0 = g * 2
            pltpu.make_async_copy(hs.at[rowv.at[b0]], buf0, sem).wait()
            pltpu.async_copy(hs.at[rowv.at[b0 + 1]], buf1, sem)
            pltpu.sync_copy(buf0, acc.at[colv.at[b0]], add=True)
            pltpu.make_async_copy(hs.at[rowv.at[b0 + 1]], buf1, sem).wait()

            @pl.when(b0 + 2 < NB1)
            def _():
                pltpu.async_copy(hs.at[rowv.at[b0 + 2]], buf0, sem)

            pltpu.sync_copy(buf1, acc.at[colv.at[b0 + 1]], add=True)
            return carry

        lax.fori_loop(0, NB1 // 2, body, 0)
        if NB1 % 2:
            b = NB1 - 1
            pltpu.make_async_copy(hs.at[rowv.at[b]], buf0, sem).wait()
            pltpu.sync_copy(buf0, acc.at[colv.at[b]], add=True)

    @pl.when(c == 0)
    def _():
        edge_loop(hs_a)

    @pl.when(c == 1)
    def _():
        edge_loop(hs_b)

    plsc.subcore_barrier()

    @pl.when(c == 0)
    def _():
        pltpu.sync_copy(acc.at[pl.ds(s * RPT, RPT)], out_a.at[pl.ds(s * RPT, RPT)])

    @pl.when(c == 1)
    def _():
        pltpu.sync_copy(acc.at[pl.ds(s * RPT, RPT)], out_b.at[pl.ds(s * RPT, RPT)])


def _make_agg_kernel(width):
    return pl.kernel(
        _agg_body,
        out_type=(
            jax.ShapeDtypeStruct((NPAD, width), jnp.float32),
            jax.ShapeDtypeStruct((NPAD, width), jnp.float32),
        ),
        mesh=_mesh(),
        scratch_types=[
            pltpu.VMEM((NB1, LB), jnp.int32),
            pltpu.VMEM((NB1, LB), jnp.int32),
            pltpu.VMEM((LB, width), jnp.float32),
            pltpu.VMEM((LB, width), jnp.float32),
            pltpu.VMEM_SHARED((NPAD, width), jnp.float32),
            pltpu.SemaphoreType.DMA,
        ],
    )


def _mm_body(x_ref, w_ref, o_ref):
    o_ref[...] = jnp.dot(x_ref[...], w_ref[...],
                         preferred_element_type=jnp.float32)


def _tc_matmul(x, w):
    m, k = x.shape
    n = w.shape[1]
    return pl.pallas_call(
        _mm_body,
        grid=(m // BLK,),
        in_specs=[
            pl.BlockSpec((BLK, k), lambda i: (i, 0)),
            pl.BlockSpec((k, n), lambda i: (0, 0)),
        ],
        out_specs=pl.BlockSpec((BLK, n), lambda i: (i, 0)),
        out_shape=jax.ShapeDtypeStruct((m, n), jnp.float32),
    )(x, w)


def _dinv_of(dega, degb):
    deg = dega[:, :1] + degb[:, :1] + 1.0
    return lax.rsqrt(deg)


def _prep1_body(h1_ref, dega_ref, degb_ref, hs1a_ref, hs1b_ref, hss1_ref):
    dinv = _dinv_of(dega_ref[...], degb_ref[...])
    hs = h1_ref[...] * dinv
    hs1a_ref[...] = hs[:, :128]
    hs1b_ref[...] = hs[:, 128:]
    hss1_ref[...] = hs * dinv


def _tc_prep1(h1, dega, degb):
    return pl.pallas_call(
        _prep1_body,
        grid=(GRID,),
        in_specs=[
            pl.BlockSpec((BLK, D_HID), lambda i: (i, 0)),
            pl.BlockSpec((BLK, 16), lambda i: (i, 0)),
            pl.BlockSpec((BLK, 16), lambda i: (i, 0)),
        ],
        out_specs=(
            pl.BlockSpec((BLK, 128), lambda i: (i, 0)),
            pl.BlockSpec((BLK, 128), lambda i: (i, 0)),
            pl.BlockSpec((BLK, D_HID), lambda i: (i, 0)),
        ),
        out_shape=(
            jax.ShapeDtypeStruct((N_NODES, 128), jnp.float32),
            jax.ShapeDtypeStruct((N_NODES, 128), jnp.float32),
            jax.ShapeDtypeStruct((N_NODES, D_HID), jnp.float32),
        ),
    )(h1, dega, degb)


def _agg1_body(s1a_ref, s1b_ref, hss1_ref, dega_ref, degb_ref, b1_ref,
               agg_ref, sums_ref, sumsq_ref):
    i = pl.program_id(0)
    dinv = _dinv_of(dega_ref[...], degb_ref[...])
    s = jnp.concatenate([s1a_ref[...], s1b_ref[...]], axis=1)
    agg = dinv * s + hss1_ref[...] + b1_ref[...]
    agg_ref[...] = agg

    @pl.when(i == 0)
    def _():
        sums_ref[...] = jnp.zeros_like(sums_ref)
        sumsq_ref[...] = jnp.zeros_like(sumsq_ref)

    sums_ref[...] += jnp.sum(agg, axis=0, keepdims=True)
    sumsq_ref[...] += jnp.sum(agg * agg, axis=0, keepdims=True)


def _tc_agg1(s1a, s1b, hss1, dega, degb, b1):
    return pl.pallas_call(
        _agg1_body,
        grid=(GRID,),
        in_specs=[
            pl.BlockSpec((BLK, 128), lambda i: (i, 0)),
            pl.BlockSpec((BLK, 128), lambda i: (i, 0)),
            pl.BlockSpec((BLK, D_HID), lambda i: (i, 0)),
            pl.BlockSpec((BLK, 16), lambda i: (i, 0)),
            pl.BlockSpec((BLK, 16), lambda i: (i, 0)),
            pl.BlockSpec((1, D_HID), lambda i: (0, 0)),
        ],
        out_specs=(
            pl.BlockSpec((BLK, D_HID), lambda i: (i, 0)),
            pl.BlockSpec((1, D_HID), lambda i: (0, 0)),
            pl.BlockSpec((1, D_HID), lambda i: (0, 0)),
        ),
        out_shape=(
            jax.ShapeDtypeStruct((N_NODES, D_HID), jnp.float32),
            jax.ShapeDtypeStruct((1, D_HID), jnp.float32),
            jax.ShapeDtypeStruct((1, D_HID), jnp.float32),
        ),
    )(s1a, s1b, hss1, dega, degb, b1)


def _bn2_body(agg_ref, sums_ref, sumsq_ref, gamma_ref, beta_ref, w2_ref,
              dega_ref, degb_ref, hs2a_ref, hs2b_ref, hss2_ref):
    inv_n = 1.0 / N_NODES
    mu = sums_ref[...] * inv_n
    var = sumsq_ref[...] * inv_n - mu * mu
    scale = gamma_ref[...] * lax.rsqrt(var + 1e-5)
    hn = (agg_ref[...] - mu) * scale + beta_ref[...]
    hn = jnp.maximum(hn, 0.0)
    h2 = jnp.dot(hn, w2_ref[...], preferred_element_type=jnp.float32)
    dinv = _dinv_of(dega_ref[...], degb_ref[...])
    hs2 = h2 * dinv
    hs2a_ref[...] = hs2[:, :32]
    hs2b_ref[...] = hs2[:, 32:]
    hss2_ref[...] = hs2 * dinv


def _tc_bn2(agg1, sums, sumsq, gamma1, beta1, w2p, dega, degb):
    return pl.pallas_call(
        _bn2_body,
        grid=(GRID,),
        in_specs=[
            pl.BlockSpec((BLK, D_HID), lambda i: (i, 0)),
            pl.BlockSpec((1, D_HID), lambda i: (0, 0)),
            pl.BlockSpec((1, D_HID), lambda i: (0, 0)),
            pl.BlockSpec((1, D_HID), lambda i: (0, 0)),
            pl.BlockSpec((1, D_HID), lambda i: (0, 0)),
            pl.BlockSpec((D_HID, D_OUT_PAD), lambda i: (0, 0)),
            pl.BlockSpec((BLK, 16), lambda i: (i, 0)),
            pl.BlockSpec((BLK, 16), lambda i: (i, 0)),
        ],
        out_specs=(
            pl.BlockSpec((BLK, 32), lambda i: (i, 0)),
            pl.BlockSpec((BLK, 32), lambda i: (i, 0)),
            pl.BlockSpec((BLK, D_OUT_PAD), lambda i: (i, 0)),
        ),
        out_shape=(
            jax.ShapeDtypeStruct((N_NODES, 32), jnp.float32),
            jax.ShapeDtypeStruct((N_NODES, 32), jnp.float32),
            jax.ShapeDtypeStruct((N_NODES, D_OUT_PAD), jnp.float32),
        ),
    )(agg1, sums, sumsq, gamma1, beta1, w2p, dega, degb)


def _fin_body(s2a_ref, s2b_ref, hss2_ref, dega_ref, degb_ref, b2_ref, o_ref):
    dinv = _dinv_of(dega_ref[...], degb_ref[...])
    s = jnp.concatenate([s2a_ref[...], s2b_ref[...]], axis=1)
    o_ref[...] = dinv * s + hss2_ref[...] + b2_ref[...]


def _tc_final(s2a, s2b, hss2, dega, degb, b2p):
    return pl.pallas_call(
        _fin_body,
        grid=(GRID,),
        in_specs=[
            pl.BlockSpec((BLK, 32), lambda i: (i, 0)),
            pl.BlockSpec((BLK, 32), lambda i: (i, 0)),
            pl.BlockSpec((BLK, D_OUT_PAD), lambda i: (i, 0)),
            pl.BlockSpec((BLK, 16), lambda i: (i, 0)),
            pl.BlockSpec((BLK, 16), lambda i: (i, 0)),
            pl.BlockSpec((1, D_OUT_PAD), lambda i: (0, 0)),
        ],
        out_specs=pl.BlockSpec((BLK, D_OUT_PAD), lambda i: (i, 0)),
        out_shape=jax.ShapeDtypeStruct((N_NODES, D_OUT_PAD), jnp.float32),
    )(s2a, s2b, hss2, dega, degb, b2p)


def kernel(x, edge_index, W1, b1, gamma1, beta1, W2, b2):
    row = edge_index[0].astype(jnp.int32)
    col = edge_index[1].astype(jnp.int32)

    pad1 = NB1 * LB - EPT
    rowp = jnp.pad(row.reshape(NS, EPT), ((0, 0), (0, pad1))).reshape(NS, NB1, LB)
    colp = jnp.pad(col.reshape(NS, EPT), ((0, 0), (0, pad1)),
                   constant_values=N_NODES).reshape(NS, NB1, LB)
    padd = NBD * LB - EPW
    cold = jnp.pad(col.reshape(NS * NC, EPW), ((0, 0), (0, padd)),
                   constant_values=N_NODES).reshape(NS * NC, NBD, LB)

    zeros16 = jnp.zeros((NPAD, 16), jnp.float32)
    ones16 = jnp.ones((LB, 16), jnp.float32)
    zeros128 = jnp.zeros((NPAD, 128), jnp.float32)
    zeros32 = jnp.zeros((NPAD, 32), jnp.float32)

    b1r = b1.reshape(1, D_HID)
    g1r = gamma1.reshape(1, D_HID)
    be1r = beta1.reshape(1, D_HID)
    w2p = jnp.pad(W2, ((0, 0), (0, D_OUT_PAD - D_OUT)))
    b2p = jnp.pad(b2, (0, D_OUT_PAD - D_OUT)).reshape(1, D_OUT_PAD)

    dega_p, degb_p = _make_deg_kernel()(cold, zeros16, ones16)
    dega = dega_p[:N_NODES]
    degb = degb_p[:N_NODES]
    h1 = _tc_matmul(x, W1)

    hs1a, hs1b, hss1 = _tc_prep1(h1, dega, degb)
    s1a_p, s1b_p = _make_agg_kernel(128)(hs1a, hs1b, rowp, colp, zeros128)
    agg1, sums, sumsq = _tc_agg1(s1a_p[:N_NODES], s1b_p[:N_NODES],
                                 hss1, dega, degb, b1r)
    hs2a, hs2b, hss2 = _tc_bn2(agg1, sums, sumsq, g1r, be1r, w2p, dega, degb)
    s2a_p, s2b_p = _make_agg_kernel(32)(hs2a, hs2b, rowp, colp, zeros32)
    out = _tc_final(s2a_p[:N_NODES], s2b_p[:N_NODES], hss2, dega, degb, b2p)
    return out[:, :D_OUT]


# speedup vs baseline: 8.6730x; 8.6730x over previous
import jax
import jax.numpy as jnp
from jax import lax
from jax.experimental import pallas as pl
from jax.experimental.pallas import tpu as pltpu
from jax.experimental.pallas import tpu_sc as plsc

N_NODES = 10000
N_EDGES = 160000
D_IN = 256
D_HID = 256
D_OUT = 40
D_PAD = 128

NS = 16
NC = 2
LB = 128
EPT = N_EDGES // NS
NB1 = 80
NCH = 2
CH1 = NB1 // NCH
EPW = N_EDGES // (NS * NC)
NBD = (EPW + LB - 1) // LB
NPAD = 10112
RPT = NPAD // NS
NDEG = 10240
DPT = NDEG // NS

BLK = 1000
GRID = N_NODES // BLK


def _mesh():
    return plsc.VectorSubcoreMesh(core_axis_name="c", subcore_axis_name="s")


def _deg_body(colp, dega, degb, colv, deg_t, redv, outv, stage, sem):
    c = lax.axis_index("c")
    s = lax.axis_index("s")
    wid = s * NC + c
    pltpu.sync_copy(colp.at[wid], colv)
    zero16 = jnp.zeros((16,), jnp.float32)

    def zbody(k, carry):
        deg_t[pl.ds(k * 16, 16)] = zero16
        return carry

    lax.fori_loop(0, NDEG // 16, zbody, 0)

    one16 = jnp.ones((16,), jnp.float32)

    def sbody(k, carry):
        idx = colv[pl.ds(k * 16, 16)]
        plsc.addupdate_scatter(deg_t, [idx], one16)
        return carry

    lax.fori_loop(0, (NBD * LB) // 16, sbody, 0)

    pltpu.sync_copy(deg_t, stage.at[s])
    plsc.subcore_barrier()

    pltpu.sync_copy(stage.at[:, pl.ds(s * DPT, DPT)], redv)

    def rbody(j, carry):
        acc = redv[0, pl.ds(j * 16, 16)]
        for t in range(1, NS):
            acc = acc + redv[t, pl.ds(j * 16, 16)]
        outv[pl.ds(j * 16, 16)] = acc
        return carry

    lax.fori_loop(0, DPT // 16, rbody, 0)

    @pl.when(c == 0)
    def _():
        pltpu.sync_copy(outv, dega.at[pl.ds(s * DPT, DPT)])

    @pl.when(c == 1)
    def _():
        pltpu.sync_copy(outv, degb.at[pl.ds(s * DPT, DPT)])


def _make_deg_kernel():
    return pl.kernel(
        _deg_body,
        out_type=(
            jax.ShapeDtypeStruct((NDEG,), jnp.float32),
            jax.ShapeDtypeStruct((NDEG,), jnp.float32),
        ),
        mesh=_mesh(),
        scratch_types=[
            pltpu.VMEM((NBD * LB,), jnp.int32),
            pltpu.VMEM((NDEG,), jnp.float32),
            pltpu.VMEM((NS, DPT), jnp.float32),
            pltpu.VMEM((DPT,), jnp.float32),
            pltpu.VMEM_SHARED((NS, NDEG), jnp.float32),
            pltpu.SemaphoreType.DMA,
        ],
        compiler_params=pltpu.CompilerParams(needs_layout_passes=False),
    )


def _edge_pipeline(hs, rowv, colv, nb, buf0, buf1, acc, sem):
    pltpu.async_copy(hs.at[rowv.at[0]], buf0, sem)

    def body(g, carry):
        b0 = g * 2
        pltpu.make_async_copy(hs.at[rowv.at[b0]], buf0, sem).wait()
        pltpu.async_copy(hs.at[rowv.at[b0 + 1]], buf1, sem)
        pltpu.sync_copy(buf0, acc.at[colv.at[b0]], add=True)
        pltpu.make_async_copy(hs.at[rowv.at[b0 + 1]], buf1, sem).wait()

        @pl.when(b0 + 2 < nb)
        def _():
            pltpu.async_copy(hs.at[rowv.at[b0 + 2]], buf0, sem)

        pltpu.sync_copy(buf1, acc.at[colv.at[b0 + 1]], add=True)
        return carry

    lax.fori_loop(0, nb // 2, body, 0)
    if nb % 2:
        b = nb - 1
        pltpu.make_async_copy(hs.at[rowv.at[b]], buf0, sem).wait()
        pltpu.sync_copy(buf0, acc.at[colv.at[b]], add=True)


def _zero_acc(s, zeros, acc):
    pltpu.sync_copy(zeros.at[pl.ds(s * RPT, RPT)], acc.at[pl.ds(s * RPT, RPT)])


def _copy_out(c, s, acc, out_a, out_b):
    @pl.when(c == 0)
    def _():
        pltpu.sync_copy(acc.at[pl.ds(s * RPT, RPT)], out_a.at[pl.ds(s * RPT, RPT)])

    @pl.when(c == 1)
    def _():
        pltpu.sync_copy(acc.at[pl.ds(s * RPT, RPT)], out_b.at[pl.ds(s * RPT, RPT)])


def _agg1_sc_body(hs_a, hs_b, rowp, colp, zeros,
                  out_a, out_b, rowv, colv, buf0, buf1, acc, sem):
    c = lax.axis_index("c")
    s = lax.axis_index("s")
    _zero_acc(s, zeros, acc)
    plsc.subcore_barrier()

    def run(hs):
        for ci in range(NCH):
            pltpu.sync_copy(rowp.at[s].at[pl.ds(ci * CH1, CH1)], rowv)
            pltpu.sync_copy(colp.at[s].at[pl.ds(ci * CH1, CH1)], colv)
            _edge_pipeline(hs, rowv, colv, CH1, buf0, buf1, acc, sem)

    @pl.when(c == 0)
    def _():
        run(hs_a)

    @pl.when(c == 1)
    def _():
        run(hs_b)

    plsc.subcore_barrier()
    _copy_out(c, s, acc, out_a, out_b)


def _make_agg1_kernel():
    return pl.kernel(
        _agg1_sc_body,
        out_type=(
            jax.ShapeDtypeStruct((NPAD, D_PAD), jnp.float32),
            jax.ShapeDtypeStruct((NPAD, D_PAD), jnp.float32),
        ),
        mesh=_mesh(),
        scratch_types=[
            pltpu.VMEM((CH1, LB), jnp.int32),
            pltpu.VMEM((CH1, LB), jnp.int32),
            pltpu.VMEM((LB, D_PAD), jnp.float32),
            pltpu.VMEM((LB, D_PAD), jnp.float32),
            pltpu.VMEM_SHARED((NPAD, D_PAD), jnp.float32),
            pltpu.SemaphoreType.DMA,
        ],
    )


def _agg2_sc_body(hs, rowp, colp, zeros,
                  out_a, out_b, rowv, colv, buf0, buf1, acc, sem):
    c = lax.axis_index("c")
    s = lax.axis_index("s")
    wid = s * NC + c
    _zero_acc(s, zeros, acc)
    pltpu.sync_copy(rowp.at[wid], rowv)
    pltpu.sync_copy(colp.at[wid], colv)
    plsc.subcore_barrier()
    _edge_pipeline(hs, rowv, colv, NBD, buf0, buf1, acc, sem)
    plsc.subcore_barrier()
    _copy_out(c, s, acc, out_a, out_b)


def _make_agg2_kernel():
    return pl.kernel(
        _agg2_sc_body,
        out_type=(
            jax.ShapeDtypeStruct((NPAD, D_PAD), jnp.float32),
            jax.ShapeDtypeStruct((NPAD, D_PAD), jnp.float32),
        ),
        mesh=_mesh(),
        scratch_types=[
            pltpu.VMEM((NBD, LB), jnp.int32),
            pltpu.VMEM((NBD, LB), jnp.int32),
            pltpu.VMEM((LB, D_PAD), jnp.float32),
            pltpu.VMEM((LB, D_PAD), jnp.float32),
            pltpu.VMEM_SHARED((NPAD, D_PAD), jnp.float32),
            pltpu.SemaphoreType.DMA,
        ],
    )


def _mm_body(x_ref, w_ref, o_ref):
    o_ref[...] = jnp.dot(x_ref[...], w_ref[...],
                         preferred_element_type=jnp.float32)


def _tc_matmul(x, w):
    m, k = x.shape
    n = w.shape[1]
    return pl.pallas_call(
        _mm_body,
        grid=(m // BLK,),
        in_specs=[
            pl.BlockSpec((BLK, k), lambda i: (i, 0)),
            pl.BlockSpec((k, n), lambda i: (0, 0)),
        ],
        out_specs=pl.BlockSpec((BLK, n), lambda i: (i, 0)),
        out_shape=jax.ShapeDtypeStruct((m, n), jnp.float32),
    )(x, w)


def _dinv_of(dega, degb):
    deg = dega + degb + 1.0
    return lax.rsqrt(deg)


def _prep1_body(h1_ref, dega_ref, degb_ref, hs1a_ref, hs1b_ref, hss1_ref):
    dinv = _dinv_of(dega_ref[...], degb_ref[...])
    hs = h1_ref[...] * dinv
    hs1a_ref[...] = hs[:, :D_PAD]
    hs1b_ref[...] = hs[:, D_PAD:]
    hss1_ref[...] = hs * dinv


def _tc_prep1(h1, dega, degb):
    return pl.pallas_call(
        _prep1_body,
        grid=(GRID,),
        in_specs=[
            pl.BlockSpec((BLK, D_HID), lambda i: (i, 0)),
            pl.BlockSpec((BLK, 1), lambda i: (i, 0)),
            pl.BlockSpec((BLK, 1), lambda i: (i, 0)),
        ],
        out_specs=(
            pl.BlockSpec((BLK, D_PAD), lambda i: (i, 0)),
            pl.BlockSpec((BLK, D_PAD), lambda i: (i, 0)),
            pl.BlockSpec((BLK, D_HID), lambda i: (i, 0)),
        ),
        out_shape=(
            jax.ShapeDtypeStruct((N_NODES, D_PAD), jnp.float32),
            jax.ShapeDtypeStruct((N_NODES, D_PAD), jnp.float32),
            jax.ShapeDtypeStruct((N_NODES, D_HID), jnp.float32),
        ),
    )(h1, dega, degb)


def _agg1_body(s1a_ref, s1b_ref, hss1_ref, dega_ref, degb_ref, b1_ref,
               agg_ref, sums_ref, sumsq_ref):
    i = pl.program_id(0)
    dinv = _dinv_of(dega_ref[...], degb_ref[...])
    s = jnp.concatenate([s1a_ref[...], s1b_ref[...]], axis=1)
    agg = dinv * s + hss1_ref[...] + b1_ref[...]
    agg_ref[...] = agg

    @pl.when(i == 0)
    def _():
        sums_ref[...] = jnp.zeros_like(sums_ref)
        sumsq_ref[...] = jnp.zeros_like(sumsq_ref)

    sums_ref[...] += jnp.sum(agg, axis=0, keepdims=True)
    sumsq_ref[...] += jnp.sum(agg * agg, axis=0, keepdims=True)


def _tc_agg1(s1a, s1b, hss1, dega, degb, b1):
    return pl.pallas_call(
        _agg1_body,
        grid=(GRID,),
        in_specs=[
            pl.BlockSpec((BLK, D_PAD), lambda i: (i, 0)),
            pl.BlockSpec((BLK, D_PAD), lambda i: (i, 0)),
            pl.BlockSpec((BLK, D_HID), lambda i: (i, 0)),
            pl.BlockSpec((BLK, 1), lambda i: (i, 0)),
            pl.BlockSpec((BLK, 1), lambda i: (i, 0)),
            pl.BlockSpec((1, D_HID), lambda i: (0, 0)),
        ],
        out_specs=(
            pl.BlockSpec((BLK, D_HID), lambda i: (i, 0)),
            pl.BlockSpec((1, D_HID), lambda i: (0, 0)),
            pl.BlockSpec((1, D_HID), lambda i: (0, 0)),
        ),
        out_shape=(
            jax.ShapeDtypeStruct((N_NODES, D_HID), jnp.float32),
            jax.ShapeDtypeStruct((1, D_HID), jnp.float32),
            jax.ShapeDtypeStruct((1, D_HID), jnp.float32),
        ),
    )(s1a, s1b, hss1, dega, degb, b1)


def _bn2_body(agg_ref, sums_ref, sumsq_ref, gamma_ref, beta_ref, w2_ref,
              dega_ref, degb_ref, hs2_ref, hss2_ref):
    inv_n = 1.0 / N_NODES
    mu = sums_ref[...] * inv_n
    var = sumsq_ref[...] * inv_n - mu * mu
    scale = gamma_ref[...] * lax.rsqrt(var + 1e-5)
    hn = (agg_ref[...] - mu) * scale + beta_ref[...]
    hn = jnp.maximum(hn, 0.0)
    h2 = jnp.dot(hn, w2_ref[...], preferred_element_type=jnp.float32)
    dinv = _dinv_of(dega_ref[...], degb_ref[...])
    hs2 = h2 * dinv
    hs2_ref[...] = hs2
    hss2_ref[...] = hs2 * dinv


def _tc_bn2(agg1, sums, sumsq, gamma1, beta1, w2p, dega, degb):
    return pl.pallas_call(
        _bn2_body,
        grid=(GRID,),
        in_specs=[
            pl.BlockSpec((BLK, D_HID), lambda i: (i, 0)),
            pl.BlockSpec((1, D_HID), lambda i: (0, 0)),
            pl.BlockSpec((1, D_HID), lambda i: (0, 0)),
            pl.BlockSpec((1, D_HID), lambda i: (0, 0)),
            pl.BlockSpec((1, D_HID), lambda i: (0, 0)),
            pl.BlockSpec((D_HID, D_PAD), lambda i: (0, 0)),
            pl.BlockSpec((BLK, 1), lambda i: (i, 0)),
            pl.BlockSpec((BLK, 1), lambda i: (i, 0)),
        ],
        out_specs=(
            pl.BlockSpec((BLK, D_PAD), lambda i: (i, 0)),
            pl.BlockSpec((BLK, D_PAD), lambda i: (i, 0)),
        ),
        out_shape=(
            jax.ShapeDtypeStruct((N_NODES, D_PAD), jnp.float32),
            jax.ShapeDtypeStruct((N_NODES, D_PAD), jnp.float32),
        ),
    )(agg1, sums, sumsq, gamma1, beta1, w2p, dega, degb)


def _fin_body(s2a_ref, s2b_ref, hss2_ref, dega_ref, degb_ref, b2_ref, o_ref):
    dinv = _dinv_of(dega_ref[...], degb_ref[...])
    s = s2a_ref[...] + s2b_ref[...]
    o_ref[...] = dinv * s + hss2_ref[...] + b2_ref[...]


def _tc_final(s2a, s2b, hss2, dega, degb, b2p):
    return pl.pallas_call(
        _fin_body,
        grid=(GRID,),
        in_specs=[
            pl.BlockSpec((BLK, D_PAD), lambda i: (i, 0)),
            pl.BlockSpec((BLK, D_PAD), lambda i: (i, 0)),
            pl.BlockSpec((BLK, D_PAD), lambda i: (i, 0)),
            pl.BlockSpec((BLK, 1), lambda i: (i, 0)),
            pl.BlockSpec((BLK, 1), lambda i: (i, 0)),
            pl.BlockSpec((1, D_PAD), lambda i: (0, 0)),
        ],
        out_specs=pl.BlockSpec((BLK, D_PAD), lambda i: (i, 0)),
        out_shape=jax.ShapeDtypeStruct((N_NODES, D_PAD), jnp.float32),
    )(s2a, s2b, hss2, dega, degb, b2p)


def kernel(x, edge_index, W1, b1, gamma1, beta1, W2, b2):
    row = edge_index[0].astype(jnp.int32)
    col = edge_index[1].astype(jnp.int32)

    pad1 = NB1 * LB - EPT
    rowp = jnp.pad(row.reshape(NS, EPT), ((0, 0), (0, pad1))).reshape(NS, NB1, LB)
    colp = jnp.pad(col.reshape(NS, EPT), ((0, 0), (0, pad1)),
                   constant_values=N_NODES).reshape(NS, NB1, LB)
    padd = NBD * LB - EPW
    rowd = jnp.pad(row.reshape(NS * NC, EPW), ((0, 0), (0, padd))
                   ).reshape(NS * NC, NBD, LB)
    cold = jnp.pad(col.reshape(NS * NC, EPW), ((0, 0), (0, padd)),
                   constant_values=N_NODES).reshape(NS * NC, NBD, LB)
    cold_flat = cold.reshape(NS * NC, NBD * LB)

    zeros128 = jnp.zeros((NPAD, D_PAD), jnp.float32)

    b1r = b1.reshape(1, D_HID)
    g1r = gamma1.reshape(1, D_HID)
    be1r = beta1.reshape(1, D_HID)
    w2p = jnp.pad(W2, ((0, 0), (0, D_PAD - D_OUT)))
    b2p = jnp.pad(b2, (0, D_PAD - D_OUT)).reshape(1, D_PAD)

    dega_p, degb_p = _make_deg_kernel()(cold_flat)
    dega = dega_p[:N_NODES].reshape(N_NODES, 1)
    degb = degb_p[:N_NODES].reshape(N_NODES, 1)
    h1 = _tc_matmul(x, W1)

    hs1a, hs1b, hss1 = _tc_prep1(h1, dega, degb)
    s1a_p, s1b_p = _make_agg1_kernel()(hs1a, hs1b, rowp, colp, zeros128)
    agg1, sums, sumsq = _tc_agg1(s1a_p[:N_NODES], s1b_p[:N_NODES],
                                 hss1, dega, degb, b1r)
    hs2, hss2 = _tc_bn2(agg1, sums, sumsq, g1r, be1r, w2p, dega, degb)
    s2a_p, s2b_p = _make_agg2_kernel()(hs2, rowd, cold, zeros128)
    out = _tc_final(s2a_p[:N_NODES], s2b_p[:N_NODES], hss2, dega, degb, b2p)
    return out[:, :D_OUT]


# speedup vs baseline: 8.9619x; 1.0333x over previous
import jax
import jax.numpy as jnp
from jax import lax
from jax.experimental import pallas as pl
from jax.experimental.pallas import tpu as pltpu
from jax.experimental.pallas import tpu_sc as plsc

N_NODES = 10000
N_EDGES = 160000
D_IN = 256
D_HID = 256
D_OUT = 40
D_PAD = 128

NS = 16
NC = 2
LB = 64
NBC = 40
EPT = N_EDGES // NS
NB1 = 160
NCH1 = NB1 // NBC
EPW = N_EDGES // (NS * NC)
NB2 = 80
NCH2 = NB2 // NBC
NBD = NB2
NPAD = 10112
RPT = NPAD // NS
NDEG = 10240
DPT = NDEG // NS

BLK = 1000
GRID = N_NODES // BLK


def _mesh():
    return plsc.VectorSubcoreMesh(core_axis_name="c", subcore_axis_name="s")


def _deg_body(colp, dega, degb, colv, deg_t, redv, outv, stage, sem):
    c = lax.axis_index("c")
    s = lax.axis_index("s")
    wid = s * NC + c
    pltpu.sync_copy(colp.at[wid], colv)
    zero16 = jnp.zeros((16,), jnp.float32)

    def zbody(k, carry):
        deg_t[pl.ds(k * 16, 16)] = zero16
        return carry

    lax.fori_loop(0, NDEG // 16, zbody, 0)

    one16 = jnp.ones((16,), jnp.float32)

    def sbody(k, carry):
        idx = colv[pl.ds(k * 16, 16)]
        plsc.addupdate_scatter(deg_t, [idx], one16)
        return carry

    lax.fori_loop(0, (NBD * LB) // 16, sbody, 0)

    pltpu.sync_copy(deg_t, stage.at[s])
    plsc.subcore_barrier()

    pltpu.sync_copy(stage.at[:, pl.ds(s * DPT, DPT)], redv)

    def rbody(j, carry):
        acc = redv[0, pl.ds(j * 16, 16)]
        for t in range(1, NS):
            acc = acc + redv[t, pl.ds(j * 16, 16)]
        outv[pl.ds(j * 16, 16)] = acc
        return carry

    lax.fori_loop(0, DPT // 16, rbody, 0)

    @pl.when(c == 0)
    def _():
        pltpu.sync_copy(outv, dega.at[pl.ds(s * DPT, DPT)])

    @pl.when(c == 1)
    def _():
        pltpu.sync_copy(outv, degb.at[pl.ds(s * DPT, DPT)])


def _make_deg_kernel():
    return pl.kernel(
        _deg_body,
        out_type=(
            jax.ShapeDtypeStruct((NDEG,), jnp.float32),
            jax.ShapeDtypeStruct((NDEG,), jnp.float32),
        ),
        mesh=_mesh(),
        scratch_types=[
            pltpu.VMEM((NBD * LB,), jnp.int32),
            pltpu.VMEM((NDEG,), jnp.float32),
            pltpu.VMEM((NS, DPT), jnp.float32),
            pltpu.VMEM((DPT,), jnp.float32),
            pltpu.VMEM_SHARED((NS, NDEG), jnp.float32),
            pltpu.SemaphoreType.DMA,
        ],
        compiler_params=pltpu.CompilerParams(needs_layout_passes=False),
    )


def _edge_pipeline(hs, rowv, colv, bufs, acc, sem_g, sem_s):
    nd = len(bufs)
    for j in range(nd):
        pltpu.async_copy(hs.at[rowv.at[j]], bufs[j], sem_g)

    def body(q, carry):
        b0 = q * nd
        for j in range(nd):
            b = b0 + j
            pltpu.make_async_copy(hs.at[rowv.at[b]], bufs[j], sem_g).wait()
            pltpu.async_copy(bufs[j], acc.at[colv.at[b]], sem_s, add=True)
        for j in range(nd):
            b = b0 + j

            @pl.when(b + nd < NBC)
            def _():
                pltpu.make_async_copy(bufs[j], acc.at[colv.at[b]], sem_s).wait()
                pltpu.async_copy(hs.at[rowv.at[b + nd]], bufs[j], sem_g)

        return carry

    lax.fori_loop(0, NBC // nd, body, 0)
    for j in range(nd):
        pltpu.make_async_copy(bufs[j], acc.at[colv.at[NBC - nd + j]], sem_s).wait()


def _zero_acc(s, zeros, acc):
    pltpu.sync_copy(zeros.at[pl.ds(s * RPT, RPT)], acc.at[pl.ds(s * RPT, RPT)])


def _copy_out(c, s, acc, out_a, out_b):
    @pl.when(c == 0)
    def _():
        pltpu.sync_copy(acc.at[pl.ds(s * RPT, RPT)], out_a.at[pl.ds(s * RPT, RPT)])

    @pl.when(c == 1)
    def _():
        pltpu.sync_copy(acc.at[pl.ds(s * RPT, RPT)], out_b.at[pl.ds(s * RPT, RPT)])


def _agg_scratch():
    return [
        pltpu.VMEM((NBC, LB), jnp.int32),
        pltpu.VMEM((NBC, LB), jnp.int32),
        pltpu.VMEM((LB, D_PAD), jnp.float32),
        pltpu.VMEM((LB, D_PAD), jnp.float32),
        pltpu.VMEM((LB, D_PAD), jnp.float32),
        pltpu.VMEM((LB, D_PAD), jnp.float32),
        pltpu.VMEM_SHARED((NPAD, D_PAD), jnp.float32),
        pltpu.SemaphoreType.DMA,
        pltpu.SemaphoreType.DMA,
    ]


def _agg1_sc_body(hs_a, hs_b, rowp, colp, zeros, out_a, out_b,
                  rowv, colv, buf0, buf1, buf2, buf3, acc, sem_g, sem_s):
    c = lax.axis_index("c")
    s = lax.axis_index("s")
    bufs = (buf0, buf1, buf2, buf3)
    _zero_acc(s, zeros, acc)
    plsc.subcore_barrier()

    def run(hs):
        for ci in range(NCH1):
            pltpu.sync_copy(rowp.at[s].at[pl.ds(ci * NBC, NBC)], rowv)
            pltpu.sync_copy(colp.at[s].at[pl.ds(ci * NBC, NBC)], colv)
            _edge_pipeline(hs, rowv, colv, bufs, acc, sem_g, sem_s)

    @pl.when(c == 0)
    def _():
        run(hs_a)

    @pl.when(c == 1)
    def _():
        run(hs_b)

    plsc.subcore_barrier()
    _copy_out(c, s, acc, out_a, out_b)


def _make_agg1_kernel():
    return pl.kernel(
        _agg1_sc_body,
        out_type=(
            jax.ShapeDtypeStruct((NPAD, D_PAD), jnp.float32),
            jax.ShapeDtypeStruct((NPAD, D_PAD), jnp.float32),
        ),
        mesh=_mesh(),
        scratch_types=_agg_scratch(),
    )


def _agg2_sc_body(hs, rowp, colp, zeros, out_a, out_b,
                  rowv, colv, buf0, buf1, buf2, buf3, acc, sem_g, sem_s):
    c = lax.axis_index("c")
    s = lax.axis_index("s")
    wid = s * NC + c
    bufs = (buf0, buf1, buf2, buf3)
    _zero_acc(s, zeros, acc)
    plsc.subcore_barrier()
    for ci in range(NCH2):
        pltpu.sync_copy(rowp.at[wid].at[pl.ds(ci * NBC, NBC)], rowv)
        pltpu.sync_copy(colp.at[wid].at[pl.ds(ci * NBC, NBC)], colv)
        _edge_pipeline(hs, rowv, colv, bufs, acc, sem_g, sem_s)
    plsc.subcore_barrier()
    _copy_out(c, s, acc, out_a, out_b)


def _make_agg2_kernel():
    return pl.kernel(
        _agg2_sc_body,
        out_type=(
            jax.ShapeDtypeStruct((NPAD, D_PAD), jnp.float32),
            jax.ShapeDtypeStruct((NPAD, D_PAD), jnp.float32),
        ),
        mesh=_mesh(),
        scratch_types=_agg_scratch(),
    )


def _mm_body(x_ref, w_ref, o_ref):
    o_ref[...] = jnp.dot(x_ref[...], w_ref[...],
                         preferred_element_type=jnp.float32)


def _tc_matmul(x, w):
    m, k = x.shape
    n = w.shape[1]
    return pl.pallas_call(
        _mm_body,
        grid=(m // BLK,),
        in_specs=[
            pl.BlockSpec((BLK, k), lambda i: (i, 0)),
            pl.BlockSpec((k, n), lambda i: (0, 0)),
        ],
        out_specs=pl.BlockSpec((BLK, n), lambda i: (i, 0)),
        out_shape=jax.ShapeDtypeStruct((m, n), jnp.float32),
    )(x, w)


def _dinv_of(dega, degb):
    deg = dega + degb + 1.0
    return lax.rsqrt(deg)


def _prep1_body(h1_ref, dega_ref, degb_ref, hs1a_ref, hs1b_ref, hss1_ref):
    dinv = _dinv_of(dega_ref[...], degb_ref[...])
    hs = h1_ref[...] * dinv
    hs1a_ref[...] = hs[:, :D_PAD]
    hs1b_ref[...] = hs[:, D_PAD:]
    hss1_ref[...] = hs * dinv


def _tc_prep1(h1, dega, degb):
    return pl.pallas_call(
        _prep1_body,
        grid=(GRID,),
        in_specs=[
            pl.BlockSpec((BLK, D_HID), lambda i: (i, 0)),
            pl.BlockSpec((BLK, 1), lambda i: (i, 0)),
            pl.BlockSpec((BLK, 1), lambda i: (i, 0)),
        ],
        out_specs=(
            pl.BlockSpec((BLK, D_PAD), lambda i: (i, 0)),
            pl.BlockSpec((BLK, D_PAD), lambda i: (i, 0)),
            pl.BlockSpec((BLK, D_HID), lambda i: (i, 0)),
        ),
        out_shape=(
            jax.ShapeDtypeStruct((N_NODES, D_PAD), jnp.float32),
            jax.ShapeDtypeStruct((N_NODES, D_PAD), jnp.float32),
            jax.ShapeDtypeStruct((N_NODES, D_HID), jnp.float32),
        ),
    )(h1, dega, degb)


def _agg1_body(s1a_ref, s1b_ref, hss1_ref, dega_ref, degb_ref, b1_ref,
               agg_ref, sums_ref, sumsq_ref):
    i = pl.program_id(0)
    dinv = _dinv_of(dega_ref[...], degb_ref[...])
    s = jnp.concatenate([s1a_ref[...], s1b_ref[...]], axis=1)
    agg = dinv * s + hss1_ref[...] + b1_ref[...]
    agg_ref[...] = agg

    @pl.when(i == 0)
    def _():
        sums_ref[...] = jnp.zeros_like(sums_ref)
        sumsq_ref[...] = jnp.zeros_like(sumsq_ref)

    sums_ref[...] += jnp.sum(agg, axis=0, keepdims=True)
    sumsq_ref[...] += jnp.sum(agg * agg, axis=0, keepdims=True)


def _tc_agg1(s1a, s1b, hss1, dega, degb, b1):
    return pl.pallas_call(
        _agg1_body,
        grid=(GRID,),
        in_specs=[
            pl.BlockSpec((BLK, D_PAD), lambda i: (i, 0)),
            pl.BlockSpec((BLK, D_PAD), lambda i: (i, 0)),
            pl.BlockSpec((BLK, D_HID), lambda i: (i, 0)),
            pl.BlockSpec((BLK, 1), lambda i: (i, 0)),
            pl.BlockSpec((BLK, 1), lambda i: (i, 0)),
            pl.BlockSpec((1, D_HID), lambda i: (0, 0)),
        ],
        out_specs=(
            pl.BlockSpec((BLK, D_HID), lambda i: (i, 0)),
            pl.BlockSpec((1, D_HID), lambda i: (0, 0)),
            pl.BlockSpec((1, D_HID), lambda i: (0, 0)),
        ),
        out_shape=(
            jax.ShapeDtypeStruct((N_NODES, D_HID), jnp.float32),
            jax.ShapeDtypeStruct((1, D_HID), jnp.float32),
            jax.ShapeDtypeStruct((1, D_HID), jnp.float32),
        ),
    )(s1a, s1b, hss1, dega, degb, b1)


def _bn2_body(agg_ref, sums_ref, sumsq_ref, gamma_ref, beta_ref, w2_ref,
              dega_ref, degb_ref, hs2_ref, hss2_ref):
    inv_n = 1.0 / N_NODES
    mu = sums_ref[...] * inv_n
    var = sumsq_ref[...] * inv_n - mu * mu
    scale = gamma_ref[...] * lax.rsqrt(var + 1e-5)
    hn = (agg_ref[...] - mu) * scale + beta_ref[...]
    hn = jnp.maximum(hn, 0.0)
    h2 = jnp.dot(hn, w2_ref[...], preferred_element_type=jnp.float32)
    dinv = _dinv_of(dega_ref[...], degb_ref[...])
    hs2 = h2 * dinv
    hs2_ref[...] = hs2
    hss2_ref[...] = hs2 * dinv


def _tc_bn2(agg1, sums, sumsq, gamma1, beta1, w2p, dega, degb):
    return pl.pallas_call(
        _bn2_body,
        grid=(GRID,),
        in_specs=[
            pl.BlockSpec((BLK, D_HID), lambda i: (i, 0)),
            pl.BlockSpec((1, D_HID), lambda i: (0, 0)),
            pl.BlockSpec((1, D_HID), lambda i: (0, 0)),
            pl.BlockSpec((1, D_HID), lambda i: (0, 0)),
            pl.BlockSpec((1, D_HID), lambda i: (0, 0)),
            pl.BlockSpec((D_HID, D_PAD), lambda i: (0, 0)),
            pl.BlockSpec((BLK, 1), lambda i: (i, 0)),
            pl.BlockSpec((BLK, 1), lambda i: (i, 0)),
        ],
        out_specs=(
            pl.BlockSpec((BLK, D_PAD), lambda i: (i, 0)),
            pl.BlockSpec((BLK, D_PAD), lambda i: (i, 0)),
        ),
        out_shape=(
            jax.ShapeDtypeStruct((N_NODES, D_PAD), jnp.float32),
            jax.ShapeDtypeStruct((N_NODES, D_PAD), jnp.float32),
        ),
    )(agg1, sums, sumsq, gamma1, beta1, w2p, dega, degb)


def _fin_body(s2a_ref, s2b_ref, hss2_ref, dega_ref, degb_ref, b2_ref, o_ref):
    dinv = _dinv_of(dega_ref[...], degb_ref[...])
    s = s2a_ref[...] + s2b_ref[...]
    o_ref[...] = dinv * s + hss2_ref[...] + b2_ref[...]


def _tc_final(s2a, s2b, hss2, dega, degb, b2p):
    return pl.pallas_call(
        _fin_body,
        grid=(GRID,),
        in_specs=[
            pl.BlockSpec((BLK, D_PAD), lambda i: (i, 0)),
            pl.BlockSpec((BLK, D_PAD), lambda i: (i, 0)),
            pl.BlockSpec((BLK, D_PAD), lambda i: (i, 0)),
            pl.BlockSpec((BLK, 1), lambda i: (i, 0)),
            pl.BlockSpec((BLK, 1), lambda i: (i, 0)),
            pl.BlockSpec((1, D_PAD), lambda i: (0, 0)),
        ],
        out_specs=pl.BlockSpec((BLK, D_PAD), lambda i: (i, 0)),
        out_shape=jax.ShapeDtypeStruct((N_NODES, D_PAD), jnp.float32),
    )(s2a, s2b, hss2, dega, degb, b2p)


def kernel(x, edge_index, W1, b1, gamma1, beta1, W2, b2):
    row = edge_index[0].astype(jnp.int32)
    col = edge_index[1].astype(jnp.int32)

    pad1 = NB1 * LB - EPT
    rowp = jnp.pad(row.reshape(NS, EPT), ((0, 0), (0, pad1))).reshape(NS, NB1, LB)
    colp = jnp.pad(col.reshape(NS, EPT), ((0, 0), (0, pad1)),
                   constant_values=N_NODES).reshape(NS, NB1, LB)
    padd = NB2 * LB - EPW
    rowd = jnp.pad(row.reshape(NS * NC, EPW), ((0, 0), (0, padd))
                   ).reshape(NS * NC, NB2, LB)
    cold = jnp.pad(col.reshape(NS * NC, EPW), ((0, 0), (0, padd)),
                   constant_values=N_NODES).reshape(NS * NC, NB2, LB)
    cold_flat = cold.reshape(NS * NC, NB2 * LB)

    zeros128 = jnp.zeros((NPAD, D_PAD), jnp.float32)

    b1r = b1.reshape(1, D_HID)
    g1r = gamma1.reshape(1, D_HID)
    be1r = beta1.reshape(1, D_HID)
    w2p = jnp.pad(W2, ((0, 0), (0, D_PAD - D_OUT)))
    b2p = jnp.pad(b2, (0, D_PAD - D_OUT)).reshape(1, D_PAD)

    dega_p, degb_p = _make_deg_kernel()(cold_flat)
    dega = dega_p[:N_NODES].reshape(N_NODES, 1)
    degb = degb_p[:N_NODES].reshape(N_NODES, 1)
    h1 = _tc_matmul(x, W1)

    hs1a, hs1b, hss1 = _tc_prep1(h1, dega, degb)
    s1a_p, s1b_p = _make_agg1_kernel()(hs1a, hs1b, rowp, colp, zeros128)
    agg1, sums, sumsq = _tc_agg1(s1a_p[:N_NODES], s1b_p[:N_NODES],
                                 hss1, dega, degb, b1r)
    hs2, hss2 = _tc_bn2(agg1, sums, sumsq, g1r, be1r, w2p, dega, degb)
    s2a_p, s2b_p = _make_agg2_kernel()(hs2, rowd, cold, zeros128)
    out = _tc_final(s2a_p[:N_NODES], s2b_p[:N_NODES], hss2, dega, degb, b2p)
    return out[:, :D_OUT]


# speedup vs baseline: 10.1609x; 1.1338x over previous
import jax
import jax.numpy as jnp
from jax import lax
from jax.experimental import pallas as pl
from jax.experimental.pallas import tpu as pltpu
from jax.experimental.pallas import tpu_sc as plsc

N_NODES = 10000
N_EDGES = 160000
D_IN = 256
D_HID = 256
D_OUT = 40
D_PAD = 128

NS = 16
NC = 2
LB = 64
NBC = 40
EPT = N_EDGES // NS
NB1 = 160
NCH1 = NB1 // NBC
EPW = N_EDGES // (NS * NC)
NB2 = 80
NCH2 = NB2 // NBC
NBD = NB2
NPAD = 10112
RPT = NPAD // NS
NDEG = 10240
DPT = NDEG // NS

BLK = 1000
GRID = N_NODES // BLK


def _mesh():
    return plsc.VectorSubcoreMesh(core_axis_name="c", subcore_axis_name="s")


def _deg_body(colp, dega, degb, colv, deg_t, redv, outv, stage, sem):
    c = lax.axis_index("c")
    s = lax.axis_index("s")
    wid = s * NC + c
    pltpu.sync_copy(colp.at[wid], colv)
    zero16 = jnp.zeros((16,), jnp.float32)

    def zbody(k, carry):
        deg_t[pl.ds(k * 16, 16)] = zero16
        return carry

    lax.fori_loop(0, NDEG // 16, zbody, 0)

    one16 = jnp.ones((16,), jnp.float32)

    def sbody(k, carry):
        idx = colv[pl.ds(k * 16, 16)]
        plsc.addupdate_scatter(deg_t, [idx], one16)
        return carry

    lax.fori_loop(0, (NBD * LB) // 16, sbody, 0)

    pltpu.sync_copy(deg_t, stage.at[s])
    plsc.subcore_barrier()

    pltpu.sync_copy(stage.at[:, pl.ds(s * DPT, DPT)], redv)

    def rbody(j, carry):
        acc = redv[0, pl.ds(j * 16, 16)]
        for t in range(1, NS):
            acc = acc + redv[t, pl.ds(j * 16, 16)]
        outv[pl.ds(j * 16, 16)] = acc
        return carry

    lax.fori_loop(0, DPT // 16, rbody, 0)

    @pl.when(c == 0)
    def _():
        pltpu.sync_copy(outv, dega.at[pl.ds(s * DPT, DPT)])

    @pl.when(c == 1)
    def _():
        pltpu.sync_copy(outv, degb.at[pl.ds(s * DPT, DPT)])


def _make_deg_kernel():
    return pl.kernel(
        _deg_body,
        out_type=(
            jax.ShapeDtypeStruct((NDEG,), jnp.float32),
            jax.ShapeDtypeStruct((NDEG,), jnp.float32),
        ),
        mesh=_mesh(),
        scratch_types=[
            pltpu.VMEM((NBD * LB,), jnp.int32),
            pltpu.VMEM((NDEG,), jnp.float32),
            pltpu.VMEM((NS, DPT), jnp.float32),
            pltpu.VMEM((DPT,), jnp.float32),
            pltpu.VMEM_SHARED((NS, NDEG), jnp.float32),
            pltpu.SemaphoreType.DMA,
        ],
        compiler_params=pltpu.CompilerParams(needs_layout_passes=False),
    )


def _edge_pipeline(hs, rowv, colv, bufs, acc, sem_g, sem_s):
    nd = len(bufs)
    for j in range(nd):
        pltpu.async_copy(hs.at[rowv.at[j]], bufs[j], sem_g)

    def body(q, carry):
        b0 = q * nd
        for j in range(nd):
            b = b0 + j
            pltpu.make_async_copy(hs.at[rowv.at[b]], bufs[j], sem_g).wait()
            pltpu.async_copy(bufs[j], acc.at[colv.at[b]], sem_s, add=True)
        for j in range(nd):
            b = b0 + j

            @pl.when(b + nd < NBC)
            def _():
                pltpu.make_async_copy(bufs[j], acc.at[colv.at[b]], sem_s).wait()
                pltpu.async_copy(hs.at[rowv.at[b + nd]], bufs[j], sem_g)

        return carry

    lax.fori_loop(0, NBC // nd, body, 0)
    for j in range(nd):
        pltpu.make_async_copy(bufs[j], acc.at[colv.at[NBC - nd + j]], sem_s).wait()


def _zero_acc(s, zeros, acc):
    pltpu.sync_copy(zeros.at[pl.ds(s * RPT, RPT)], acc.at[pl.ds(s * RPT, RPT)])


def _copy_out(c, s, acc, out_a, out_b):
    @pl.when(c == 0)
    def _():
        pltpu.sync_copy(acc.at[pl.ds(s * RPT, RPT)], out_a.at[pl.ds(s * RPT, RPT)])

    @pl.when(c == 1)
    def _():
        pltpu.sync_copy(acc.at[pl.ds(s * RPT, RPT)], out_b.at[pl.ds(s * RPT, RPT)])


def _agg_scratch():
    return [
        pltpu.VMEM((NBC, LB), jnp.int32),
        pltpu.VMEM((NBC, LB), jnp.int32),
        pltpu.VMEM((LB, D_PAD), jnp.float32),
        pltpu.VMEM((LB, D_PAD), jnp.float32),
        pltpu.VMEM((LB, D_PAD), jnp.float32),
        pltpu.VMEM((LB, D_PAD), jnp.float32),
        pltpu.VMEM_SHARED((NPAD, D_PAD), jnp.float32),
        pltpu.SemaphoreType.DMA,
        pltpu.SemaphoreType.DMA,
    ]


def _agg1_sc_body(hs_a, hs_b, rowp, colp, zeros, out_a, out_b,
                  rowv, colv, buf0, buf1, buf2, buf3, acc, sem_g, sem_s):
    c = lax.axis_index("c")
    s = lax.axis_index("s")
    bufs = (buf0, buf1, buf2, buf3)
    _zero_acc(s, zeros, acc)
    plsc.subcore_barrier()

    def run(hs):
        for ci in range(NCH1):
            pltpu.sync_copy(rowp.at[s].at[pl.ds(ci * NBC, NBC)], rowv)
            pltpu.sync_copy(colp.at[s].at[pl.ds(ci * NBC, NBC)], colv)
            _edge_pipeline(hs, rowv, colv, bufs, acc, sem_g, sem_s)

    @pl.when(c == 0)
    def _():
        run(hs_a)

    @pl.when(c == 1)
    def _():
        run(hs_b)

    plsc.subcore_barrier()
    _copy_out(c, s, acc, out_a, out_b)


def _make_agg1_kernel():
    return pl.kernel(
        _agg1_sc_body,
        out_type=(
            jax.ShapeDtypeStruct((NPAD, D_PAD), jnp.float32),
            jax.ShapeDtypeStruct((NPAD, D_PAD), jnp.float32),
        ),
        mesh=_mesh(),
        scratch_types=_agg_scratch(),
    )


def _agg2_sc_body(hs_a, hs_b, rowp, colp, zeros, out_a, out_b,
                  rowv, colv, buf0, buf1, buf2, buf3, acc, sem_g, sem_s):
    c = lax.axis_index("c")
    s = lax.axis_index("s")
    wid = s * NC + c
    bufs = (buf0, buf1, buf2, buf3)
    _zero_acc(s, zeros, acc)
    plsc.subcore_barrier()

    def run(hs):
        for ci in range(NCH2):
            pltpu.sync_copy(rowp.at[wid].at[pl.ds(ci * NBC, NBC)], rowv)
            pltpu.sync_copy(colp.at[wid].at[pl.ds(ci * NBC, NBC)], colv)
            _edge_pipeline(hs, rowv, colv, bufs, acc, sem_g, sem_s)

    @pl.when(c == 0)
    def _():
        run(hs_a)

    @pl.when(c == 1)
    def _():
        run(hs_b)

    plsc.subcore_barrier()
    _copy_out(c, s, acc, out_a, out_b)


def _make_agg2_kernel():
    return pl.kernel(
        _agg2_sc_body,
        out_type=(
            jax.ShapeDtypeStruct((NPAD, D_PAD), jnp.float32),
            jax.ShapeDtypeStruct((NPAD, D_PAD), jnp.float32),
        ),
        mesh=_mesh(),
        scratch_types=_agg_scratch(),
    )


def _mm_body(x_ref, w_ref, o_ref):
    o_ref[...] = jnp.dot(x_ref[...], w_ref[...],
                         preferred_element_type=jnp.float32)


def _tc_matmul(x, w):
    m, k = x.shape
    n = w.shape[1]
    return pl.pallas_call(
        _mm_body,
        grid=(m // BLK,),
        in_specs=[
            pl.BlockSpec((BLK, k), lambda i: (i, 0)),
            pl.BlockSpec((k, n), lambda i: (0, 0)),
        ],
        out_specs=pl.BlockSpec((BLK, n), lambda i: (i, 0)),
        out_shape=jax.ShapeDtypeStruct((m, n), jnp.float32),
    )(x, w)


def _dinv_of(dega, degb):
    deg = dega + degb + 1.0
    return lax.rsqrt(deg)


def _prep1_body(h1_ref, dega_ref, degb_ref, hs1a_ref, hs1b_ref, hss1_ref):
    dinv = _dinv_of(dega_ref[...], degb_ref[...])
    hs = h1_ref[...] * dinv
    hs1a_ref[...] = hs[:, :D_PAD]
    hs1b_ref[...] = hs[:, D_PAD:]
    hss1_ref[...] = hs * dinv


def _tc_prep1(h1, dega, degb):
    return pl.pallas_call(
        _prep1_body,
        grid=(GRID,),
        in_specs=[
            pl.BlockSpec((BLK, D_HID), lambda i: (i, 0)),
            pl.BlockSpec((BLK, 1), lambda i: (i, 0)),
            pl.BlockSpec((BLK, 1), lambda i: (i, 0)),
        ],
        out_specs=(
            pl.BlockSpec((BLK, D_PAD), lambda i: (i, 0)),
            pl.BlockSpec((BLK, D_PAD), lambda i: (i, 0)),
            pl.BlockSpec((BLK, D_HID), lambda i: (i, 0)),
        ),
        out_shape=(
            jax.ShapeDtypeStruct((N_NODES, D_PAD), jnp.float32),
            jax.ShapeDtypeStruct((N_NODES, D_PAD), jnp.float32),
            jax.ShapeDtypeStruct((N_NODES, D_HID), jnp.float32),
        ),
    )(h1, dega, degb)


def _agg1_body(s1a_ref, s1b_ref, hss1_ref, dega_ref, degb_ref, b1_ref,
               agg_ref, sums_ref, sumsq_ref):
    i = pl.program_id(0)
    dinv = _dinv_of(dega_ref[...], degb_ref[...])
    s = jnp.concatenate([s1a_ref[...], s1b_ref[...]], axis=1)
    agg = dinv * s + hss1_ref[...] + b1_ref[...]
    agg_ref[...] = agg

    @pl.when(i == 0)
    def _():
        sums_ref[...] = jnp.zeros_like(sums_ref)
        sumsq_ref[...] = jnp.zeros_like(sumsq_ref)

    sums_ref[...] += jnp.sum(agg, axis=0, keepdims=True)
    sumsq_ref[...] += jnp.sum(agg * agg, axis=0, keepdims=True)


def _tc_agg1(s1a, s1b, hss1, dega, degb, b1):
    return pl.pallas_call(
        _agg1_body,
        grid=(GRID,),
        in_specs=[
            pl.BlockSpec((BLK, D_PAD), lambda i: (i, 0)),
            pl.BlockSpec((BLK, D_PAD), lambda i: (i, 0)),
            pl.BlockSpec((BLK, D_HID), lambda i: (i, 0)),
            pl.BlockSpec((BLK, 1), lambda i: (i, 0)),
            pl.BlockSpec((BLK, 1), lambda i: (i, 0)),
            pl.BlockSpec((1, D_HID), lambda i: (0, 0)),
        ],
        out_specs=(
            pl.BlockSpec((BLK, D_HID), lambda i: (i, 0)),
            pl.BlockSpec((1, D_HID), lambda i: (0, 0)),
            pl.BlockSpec((1, D_HID), lambda i: (0, 0)),
        ),
        out_shape=(
            jax.ShapeDtypeStruct((N_NODES, D_HID), jnp.float32),
            jax.ShapeDtypeStruct((1, D_HID), jnp.float32),
            jax.ShapeDtypeStruct((1, D_HID), jnp.float32),
        ),
    )(s1a, s1b, hss1, dega, degb, b1)


def _bn2_body(agg_ref, sums_ref, sumsq_ref, gamma_ref, beta_ref, w2_ref,
              dega_ref, degb_ref, hs2a_ref, hs2b_ref, hss2_ref):
    inv_n = 1.0 / N_NODES
    mu = sums_ref[...] * inv_n
    var = sumsq_ref[...] * inv_n - mu * mu
    scale = gamma_ref[...] * lax.rsqrt(var + 1e-5)
    hn = (agg_ref[...] - mu) * scale + beta_ref[...]
    hn = jnp.maximum(hn, 0.0)
    h2 = jnp.dot(hn, w2_ref[...], preferred_element_type=jnp.float32)
    dinv = _dinv_of(dega_ref[...], degb_ref[...])
    hs2 = h2 * dinv
    hs2a_ref[...] = hs2
    hs2b_ref[...] = hs2
    hss2_ref[...] = hs2 * dinv


def _tc_bn2(agg1, sums, sumsq, gamma1, beta1, w2p, dega, degb):
    return pl.pallas_call(
        _bn2_body,
        grid=(GRID,),
        in_specs=[
            pl.BlockSpec((BLK, D_HID), lambda i: (i, 0)),
            pl.BlockSpec((1, D_HID), lambda i: (0, 0)),
            pl.BlockSpec((1, D_HID), lambda i: (0, 0)),
            pl.BlockSpec((1, D_HID), lambda i: (0, 0)),
            pl.BlockSpec((1, D_HID), lambda i: (0, 0)),
            pl.BlockSpec((D_HID, D_PAD), lambda i: (0, 0)),
            pl.BlockSpec((BLK, 1), lambda i: (i, 0)),
            pl.BlockSpec((BLK, 1), lambda i: (i, 0)),
        ],
        out_specs=(
            pl.BlockSpec((BLK, D_PAD), lambda i: (i, 0)),
            pl.BlockSpec((BLK, D_PAD), lambda i: (i, 0)),
            pl.BlockSpec((BLK, D_PAD), lambda i: (i, 0)),
        ),
        out_shape=(
            jax.ShapeDtypeStruct((N_NODES, D_PAD), jnp.float32),
            jax.ShapeDtypeStruct((N_NODES, D_PAD), jnp.float32),
            jax.ShapeDtypeStruct((N_NODES, D_PAD), jnp.float32),
        ),
    )(agg1, sums, sumsq, gamma1, beta1, w2p, dega, degb)


def _fin_body(s2a_ref, s2b_ref, hss2_ref, dega_ref, degb_ref, b2_ref, o_ref):
    dinv = _dinv_of(dega_ref[...], degb_ref[...])
    s = s2a_ref[...] + s2b_ref[...]
    o_ref[...] = dinv * s + hss2_ref[...] + b2_ref[...]


def _tc_final(s2a, s2b, hss2, dega, degb, b2p):
    return pl.pallas_call(
        _fin_body,
        grid=(GRID,),
        in_specs=[
            pl.BlockSpec((BLK, D_PAD), lambda i: (i, 0)),
            pl.BlockSpec((BLK, D_PAD), lambda i: (i, 0)),
            pl.BlockSpec((BLK, D_PAD), lambda i: (i, 0)),
            pl.BlockSpec((BLK, 1), lambda i: (i, 0)),
            pl.BlockSpec((BLK, 1), lambda i: (i, 0)),
            pl.BlockSpec((1, D_PAD), lambda i: (0, 0)),
        ],
        out_specs=pl.BlockSpec((BLK, D_PAD), lambda i: (i, 0)),
        out_shape=jax.ShapeDtypeStruct((N_NODES, D_PAD), jnp.float32),
    )(s2a, s2b, hss2, dega, degb, b2p)


def kernel(x, edge_index, W1, b1, gamma1, beta1, W2, b2):
    row = edge_index[0].astype(jnp.int32)
    col = edge_index[1].astype(jnp.int32)

    pad1 = NB1 * LB - EPT
    rowp = jnp.pad(row.reshape(NS, EPT), ((0, 0), (0, pad1))).reshape(NS, NB1, LB)
    colp = jnp.pad(col.reshape(NS, EPT), ((0, 0), (0, pad1)),
                   constant_values=N_NODES).reshape(NS, NB1, LB)
    padd = NB2 * LB - EPW
    rowd = jnp.pad(row.reshape(NS * NC, EPW), ((0, 0), (0, padd))
                   ).reshape(NS * NC, NB2, LB)
    cold = jnp.pad(col.reshape(NS * NC, EPW), ((0, 0), (0, padd)),
                   constant_values=N_NODES).reshape(NS * NC, NB2, LB)
    cold_flat = cold.reshape(NS * NC, NB2 * LB)

    zeros128 = jnp.zeros((NPAD, D_PAD), jnp.float32)

    b1r = b1.reshape(1, D_HID)
    g1r = gamma1.reshape(1, D_HID)
    be1r = beta1.reshape(1, D_HID)
    w2p = jnp.pad(W2, ((0, 0), (0, D_PAD - D_OUT)))
    b2p = jnp.pad(b2, (0, D_PAD - D_OUT)).reshape(1, D_PAD)

    dega_p, degb_p = _make_deg_kernel()(cold_flat)
    dega = dega_p[:N_NODES].reshape(N_NODES, 1)
    degb = degb_p[:N_NODES].reshape(N_NODES, 1)
    h1 = _tc_matmul(x, W1)

    hs1a, hs1b, hss1 = _tc_prep1(h1, dega, degb)
    s1a_p, s1b_p = _make_agg1_kernel()(hs1a, hs1b, rowp, colp, zeros128)
    agg1, sums, sumsq = _tc_agg1(s1a_p[:N_NODES], s1b_p[:N_NODES],
                                 hss1, dega, degb, b1r)
    hs2a, hs2b, hss2 = _tc_bn2(agg1, sums, sumsq, g1r, be1r, w2p, dega, degb)
    s2a_p, s2b_p = _make_agg2_kernel()(hs2a, hs2b, rowd, cold, zeros128)
    out = _tc_final(s2a_p[:N_NODES], s2b_p[:N_NODES], hss2, dega, degb, b2p)
    return out[:, :D_OUT]


# speedup vs baseline: 10.6802x; 1.0511x over previous
import jax
import jax.numpy as jnp
from jax import lax
from jax.experimental import pallas as pl
from jax.experimental.pallas import tpu as pltpu
from jax.experimental.pallas import tpu_sc as plsc

N_NODES = 10000
N_EDGES = 160000
D_IN = 256
D_HID = 256
D_OUT = 40
D_PAD = 128

NS = 16
NC = 2
LB = 64
NBC = 40
EPT = N_EDGES // NS
NB1 = 160
NCH1 = NB1 // NBC
EPW = N_EDGES // (NS * NC)
NB2 = 80
NCH2 = NB2 // NBC
NBD = NB2
NPAD = 10112
RPT = NPAD // NS
NDEG = 10240
DPT = NDEG // NS

BLK = 1000
GRID = N_NODES // BLK


def _mesh():
    return plsc.VectorSubcoreMesh(core_axis_name="c", subcore_axis_name="s")


def _deg_body(colp, dega, degb, colv, deg_t, redv, outv, stage, sem):
    c = lax.axis_index("c")
    s = lax.axis_index("s")
    wid = s * NC + c
    pltpu.sync_copy(colp.at[wid], colv)
    zero16 = jnp.zeros((16,), jnp.float32)

    def zbody(k, carry):
        deg_t[pl.ds(k * 16, 16)] = zero16
        return carry

    lax.fori_loop(0, NDEG // 16, zbody, 0)

    one16 = jnp.ones((16,), jnp.float32)

    def sbody(k, carry):
        idx = colv[pl.ds(k * 16, 16)]
        plsc.addupdate_scatter(deg_t, [idx], one16)
        return carry

    lax.fori_loop(0, (NBD * LB) // 16, sbody, 0)

    pltpu.sync_copy(deg_t, stage.at[s])
    plsc.subcore_barrier()

    pltpu.sync_copy(stage.at[:, pl.ds(s * DPT, DPT)], redv)

    def rbody(j, carry):
        acc = redv[0, pl.ds(j * 16, 16)]
        for t in range(1, NS):
            acc = acc + redv[t, pl.ds(j * 16, 16)]
        outv[pl.ds(j * 16, 16)] = acc
        return carry

    lax.fori_loop(0, DPT // 16, rbody, 0)

    @pl.when(c == 0)
    def _():
        pltpu.sync_copy(outv, dega.at[pl.ds(s * DPT, DPT)])

    @pl.when(c == 1)
    def _():
        pltpu.sync_copy(outv, degb.at[pl.ds(s * DPT, DPT)])


def _make_deg_kernel():
    return pl.kernel(
        _deg_body,
        out_type=(
            jax.ShapeDtypeStruct((NDEG,), jnp.float32),
            jax.ShapeDtypeStruct((NDEG,), jnp.float32),
        ),
        mesh=_mesh(),
        scratch_types=[
            pltpu.VMEM((NBD * LB,), jnp.int32),
            pltpu.VMEM((NDEG,), jnp.float32),
            pltpu.VMEM((NS, DPT), jnp.float32),
            pltpu.VMEM((DPT,), jnp.float32),
            pltpu.VMEM_SHARED((NS, NDEG), jnp.float32),
            pltpu.SemaphoreType.DMA,
        ],
        compiler_params=pltpu.CompilerParams(needs_layout_passes=False),
    )


def _edge_pipeline(hs, rowv, colv, bufs, acc, sem_g, sem_s):
    nd = len(bufs)
    for j in range(nd):
        pltpu.async_copy(hs.at[rowv.at[j]], bufs[j], sem_g)

    def body(q, carry):
        b0 = q * nd
        for j in range(nd):
            b = b0 + j
            pltpu.make_async_copy(hs.at[rowv.at[b]], bufs[j], sem_g).wait()
            pltpu.async_copy(bufs[j], acc.at[colv.at[b]], sem_s, add=True)
        for j in range(nd):
            b = b0 + j

            @pl.when(b + nd < NBC)
            def _():
                pltpu.make_async_copy(bufs[j], acc.at[colv.at[b]], sem_s).wait()
                pltpu.async_copy(hs.at[rowv.at[b + nd]], bufs[j], sem_g)

        return carry

    lax.fori_loop(0, NBC // nd, body, 0)
    for j in range(nd):
        pltpu.make_async_copy(bufs[j], acc.at[colv.at[NBC - nd + j]], sem_s).wait()


def _zero_acc(s, zeros, acc):
    pltpu.sync_copy(zeros.at[pl.ds(s * RPT, RPT)], acc.at[pl.ds(s * RPT, RPT)])


def _copy_out(c, s, acc, out_a, out_b):
    @pl.when(c == 0)
    def _():
        pltpu.sync_copy(acc.at[pl.ds(s * RPT, RPT)], out_a.at[pl.ds(s * RPT, RPT)])

    @pl.when(c == 1)
    def _():
        pltpu.sync_copy(acc.at[pl.ds(s * RPT, RPT)], out_b.at[pl.ds(s * RPT, RPT)])


def _agg_scratch():
    return [
        pltpu.VMEM((NBC, LB), jnp.int32),
        pltpu.VMEM((NBC, LB), jnp.int32),
        pltpu.VMEM((LB, D_PAD), jnp.float32),
        pltpu.VMEM((LB, D_PAD), jnp.float32),
        pltpu.VMEM((LB, D_PAD), jnp.float32),
        pltpu.VMEM((LB, D_PAD), jnp.float32),
        pltpu.VMEM_SHARED((NPAD, D_PAD), jnp.float32),
        pltpu.SemaphoreType.DMA,
        pltpu.SemaphoreType.DMA,
    ]


def _agg1_sc_body(hs_a, hs_b, rowp, colp, zeros, out_a, out_b,
                  rowv, colv, buf0, buf1, buf2, buf3, acc, sem_g, sem_s):
    c = lax.axis_index("c")
    s = lax.axis_index("s")
    bufs = (buf0, buf1, buf2, buf3)
    _zero_acc(s, zeros, acc)
    plsc.subcore_barrier()

    def run(hs):
        for ci in range(NCH1):
            pltpu.sync_copy(rowp.at[s].at[pl.ds(ci * NBC, NBC)], rowv)
            pltpu.sync_copy(colp.at[s].at[pl.ds(ci * NBC, NBC)], colv)
            _edge_pipeline(hs, rowv, colv, bufs, acc, sem_g, sem_s)

    @pl.when(c == 0)
    def _():
        run(hs_a)

    @pl.when(c == 1)
    def _():
        run(hs_b)

    plsc.subcore_barrier()
    _copy_out(c, s, acc, out_a, out_b)


def _make_agg1_kernel():
    return pl.kernel(
        _agg1_sc_body,
        out_type=(
            jax.ShapeDtypeStruct((NPAD, D_PAD), jnp.float32),
            jax.ShapeDtypeStruct((NPAD, D_PAD), jnp.float32),
        ),
        mesh=_mesh(),
        scratch_types=_agg_scratch(),
    )


def _agg2_sc_body(hs_a, hs_b, rowp, colp, zeros, out_a, out_b,
                  rowv, colv, buf0, buf1, buf2, buf3, acc, sem_g, sem_s):
    c = lax.axis_index("c")
    s = lax.axis_index("s")
    wid = s * NC + c
    bufs = (buf0, buf1, buf2, buf3)
    _zero_acc(s, zeros, acc)
    plsc.subcore_barrier()

    def run(hs):
        for ci in range(NCH2):
            pltpu.sync_copy(rowp.at[wid].at[pl.ds(ci * NBC, NBC)], rowv)
            pltpu.sync_copy(colp.at[wid].at[pl.ds(ci * NBC, NBC)], colv)
            _edge_pipeline(hs, rowv, colv, bufs, acc, sem_g, sem_s)

    @pl.when(c == 0)
    def _():
        run(hs_a)

    @pl.when(c == 1)
    def _():
        run(hs_b)

    plsc.subcore_barrier()
    _copy_out(c, s, acc, out_a, out_b)


def _make_agg2_kernel():
    return pl.kernel(
        _agg2_sc_body,
        out_type=(
            jax.ShapeDtypeStruct((NPAD, D_PAD), jnp.float32),
            jax.ShapeDtypeStruct((NPAD, D_PAD), jnp.float32),
        ),
        mesh=_mesh(),
        scratch_types=_agg_scratch(),
    )


def _dinv_of(dega, degb):
    deg = dega + degb + 1.0
    return lax.rsqrt(deg)


def _mmprep_body(x_ref, w_ref, dega_ref, degb_ref,
                 hs1a_ref, hs1b_ref, hss1_ref):
    h1 = jnp.dot(x_ref[...], w_ref[...], preferred_element_type=jnp.float32)
    dinv = _dinv_of(dega_ref[...], degb_ref[...])
    hs = h1 * dinv
    hs1a_ref[...] = hs[:, :D_PAD]
    hs1b_ref[...] = hs[:, D_PAD:]
    hss1_ref[...] = hs * dinv


def _tc_mmprep(x, w1, dega, degb):
    return pl.pallas_call(
        _mmprep_body,
        grid=(GRID,),
        in_specs=[
            pl.BlockSpec((BLK, D_IN), lambda i: (i, 0)),
            pl.BlockSpec((D_IN, D_HID), lambda i: (0, 0)),
            pl.BlockSpec((BLK, 1), lambda i: (i, 0)),
            pl.BlockSpec((BLK, 1), lambda i: (i, 0)),
        ],
        out_specs=(
            pl.BlockSpec((BLK, D_PAD), lambda i: (i, 0)),
            pl.BlockSpec((BLK, D_PAD), lambda i: (i, 0)),
            pl.BlockSpec((BLK, D_HID), lambda i: (i, 0)),
        ),
        out_shape=(
            jax.ShapeDtypeStruct((N_NODES, D_PAD), jnp.float32),
            jax.ShapeDtypeStruct((N_NODES, D_PAD), jnp.float32),
            jax.ShapeDtypeStruct((N_NODES, D_HID), jnp.float32),
        ),
    )(x, w1, dega, degb)


def _mid_body(s1a_ref, s1b_ref, hss1_ref, dega_ref, degb_ref, b1_ref,
              gamma_ref, beta_ref, w2_ref,
              hs2a_ref, hs2b_ref, hss2_ref, agg_ref, sums_ref, sumsq_ref):
    p = pl.program_id(0)
    j = pl.program_id(1)
    dinv = _dinv_of(dega_ref[...], degb_ref[...])

    @pl.when(p == 0)
    def _():
        s = jnp.concatenate([s1a_ref[...], s1b_ref[...]], axis=1)
        agg = dinv * s + hss1_ref[...] + b1_ref[...]
        agg_ref[pl.ds(j * BLK, BLK), :] = agg

        @pl.when(j == 0)
        def _():
            sums_ref[...] = jnp.zeros_like(sums_ref)
            sumsq_ref[...] = jnp.zeros_like(sumsq_ref)

        sums_ref[...] += jnp.sum(agg, axis=0, keepdims=True)
        sumsq_ref[...] += jnp.sum(agg * agg, axis=0, keepdims=True)

    @pl.when(p == 1)
    def _():
        inv_n = 1.0 / N_NODES
        mu = sums_ref[...] * inv_n
        var = sumsq_ref[...] * inv_n - mu * mu
        scale = gamma_ref[...] * lax.rsqrt(var + 1e-5)
        hn = (agg_ref[pl.ds(j * BLK, BLK), :] - mu) * scale + beta_ref[...]
        hn = jnp.maximum(hn, 0.0)
        h2 = jnp.dot(hn, w2_ref[...], preferred_element_type=jnp.float32)
        hs2 = h2 * dinv
        hs2a_ref[...] = hs2
        hs2b_ref[...] = hs2
        hss2_ref[...] = hs2 * dinv


def _tc_mid(s1a, s1b, hss1, dega, degb, b1, gamma1, beta1, w2p):
    def rowmap(p, j):
        return (j, 0)

    def constmap(p, j):
        return (0, 0)

    def outmap(p, j):
        return (j * p, 0)

    return pl.pallas_call(
        _mid_body,
        grid=(2, GRID),
        in_specs=[
            pl.BlockSpec((BLK, D_PAD), rowmap),
            pl.BlockSpec((BLK, D_PAD), rowmap),
            pl.BlockSpec((BLK, D_HID), rowmap),
            pl.BlockSpec((BLK, 1), rowmap),
            pl.BlockSpec((BLK, 1), rowmap),
            pl.BlockSpec((1, D_HID), constmap),
            pl.BlockSpec((1, D_HID), constmap),
            pl.BlockSpec((1, D_HID), constmap),
            pl.BlockSpec((D_HID, D_PAD), constmap),
        ],
        out_specs=(
            pl.BlockSpec((BLK, D_PAD), outmap),
            pl.BlockSpec((BLK, D_PAD), outmap),
            pl.BlockSpec((BLK, D_PAD), outmap),
        ),
        out_shape=(
            jax.ShapeDtypeStruct((N_NODES, D_PAD), jnp.float32),
            jax.ShapeDtypeStruct((N_NODES, D_PAD), jnp.float32),
            jax.ShapeDtypeStruct((N_NODES, D_PAD), jnp.float32),
        ),
        scratch_shapes=[
            pltpu.VMEM((N_NODES, D_HID), jnp.float32),
            pltpu.VMEM((1, D_HID), jnp.float32),
            pltpu.VMEM((1, D_HID), jnp.float32),
        ],
    )(s1a, s1b, hss1, dega, degb, b1, gamma1, beta1, w2p)


def _fin_body(s2a_ref, s2b_ref, hss2_ref, dega_ref, degb_ref, b2_ref, o_ref):
    dinv = _dinv_of(dega_ref[...], degb_ref[...])
    s = s2a_ref[...] + s2b_ref[...]
    o_ref[...] = (dinv * s + hss2_ref[...] + b2_ref[...])[:, :D_OUT]


def _tc_final(s2a, s2b, hss2, dega, degb, b2p):
    return pl.pallas_call(
        _fin_body,
        grid=(GRID,),
        in_specs=[
            pl.BlockSpec((BLK, D_PAD), lambda i: (i, 0)),
            pl.BlockSpec((BLK, D_PAD), lambda i: (i, 0)),
            pl.BlockSpec((BLK, D_PAD), lambda i: (i, 0)),
            pl.BlockSpec((BLK, 1), lambda i: (i, 0)),
            pl.BlockSpec((BLK, 1), lambda i: (i, 0)),
            pl.BlockSpec((1, D_PAD), lambda i: (0, 0)),
        ],
        out_specs=pl.BlockSpec((BLK, D_OUT), lambda i: (i, 0)),
        out_shape=jax.ShapeDtypeStruct((N_NODES, D_OUT), jnp.float32),
    )(s2a, s2b, hss2, dega, degb, b2p)


def kernel(x, edge_index, W1, b1, gamma1, beta1, W2, b2):
    row = edge_index[0].astype(jnp.int32)
    col = edge_index[1].astype(jnp.int32)

    pad1 = NB1 * LB - EPT
    rowp = jnp.pad(row.reshape(NS, EPT), ((0, 0), (0, pad1))).reshape(NS, NB1, LB)
    colp = jnp.pad(col.reshape(NS, EPT), ((0, 0), (0, pad1)),
                   constant_values=N_NODES).reshape(NS, NB1, LB)
    padd = NB2 * LB - EPW
    rowd = jnp.pad(row.reshape(NS * NC, EPW), ((0, 0), (0, padd))
                   ).reshape(NS * NC, NB2, LB)
    cold = jnp.pad(col.reshape(NS * NC, EPW), ((0, 0), (0, padd)),
                   constant_values=N_NODES).reshape(NS * NC, NB2, LB)
    cold_flat = cold.reshape(NS * NC, NB2 * LB)

    zeros128 = jnp.zeros((NPAD, D_PAD), jnp.float32)

    b1r = b1.reshape(1, D_HID)
    g1r = gamma1.reshape(1, D_HID)
    be1r = beta1.reshape(1, D_HID)
    w2p = jnp.pad(W2, ((0, 0), (0, D_PAD - D_OUT)))
    b2p = jnp.pad(b2, (0, D_PAD - D_OUT)).reshape(1, D_PAD)

    dega_p, degb_p = _make_deg_kernel()(cold_flat)
    dega = dega_p.reshape(NDEG, 1)
    degb = degb_p.reshape(NDEG, 1)

    hs1a, hs1b, hss1 = _tc_mmprep(x, W1, dega, degb)
    s1a_p, s1b_p = _make_agg1_kernel()(hs1a, hs1b, rowp, colp, zeros128)
    hs2a, hs2b, hss2 = _tc_mid(s1a_p, s1b_p, hss1, dega, degb,
                               b1r, g1r, be1r, w2p)
    s2a_p, s2b_p = _make_agg2_kernel()(hs2a, hs2b, rowd, cold, zeros128)
    return _tc_final(s2a_p, s2b_p, hss2, dega, degb, b2p)


# speedup vs baseline: 11.7356x; 1.0988x over previous
import jax
import jax.numpy as jnp
from jax import lax
from jax.experimental import pallas as pl
from jax.experimental.pallas import tpu as pltpu
from jax.experimental.pallas import tpu_sc as plsc

N_NODES = 10000
N_EDGES = 160000
D_IN = 256
D_HID = 256
D_OUT = 40
D_PAD = 128
D_OUT2 = 64

NS = 16
NC = 2
LB = 64
NBC = 40
EPT = N_EDGES // NS
NB1 = 160
NCH1 = NB1 // NBC
EPW = N_EDGES // (NS * NC)
NB2 = 80
NCH2 = NB2 // NBC
NBD = NB2
NPAD = 10112
RPT = NPAD // NS
NDEG = 10240
DPT = NDEG // NS

BLK = 1000
GRID = N_NODES // BLK


def _mesh():
    return plsc.VectorSubcoreMesh(core_axis_name="c", subcore_axis_name="s")


def _deg_body(colp, dega, degb, colv, deg_t, redv, outv, stage, sem):
    c = lax.axis_index("c")
    s = lax.axis_index("s")
    wid = s * NC + c
    pltpu.sync_copy(colp.at[wid], colv)
    zero16 = jnp.zeros((16,), jnp.float32)

    def zbody(k, carry):
        deg_t[pl.ds(k * 16, 16)] = zero16
        return carry

    lax.fori_loop(0, NDEG // 16, zbody, 0)

    one16 = jnp.ones((16,), jnp.float32)

    def sbody(k, carry):
        idx = colv[pl.ds(k * 16, 16)]
        plsc.addupdate_scatter(deg_t, [idx], one16)
        return carry

    lax.fori_loop(0, (NBD * LB) // 16, sbody, 0)

    pltpu.sync_copy(deg_t, stage.at[s])
    plsc.subcore_barrier()

    pltpu.sync_copy(stage.at[:, pl.ds(s * DPT, DPT)], redv)

    def rbody(j, carry):
        acc = redv[0, pl.ds(j * 16, 16)]
        for t in range(1, NS):
            acc = acc + redv[t, pl.ds(j * 16, 16)]
        outv[pl.ds(j * 16, 16)] = acc
        return carry

    lax.fori_loop(0, DPT // 16, rbody, 0)

    @pl.when(c == 0)
    def _():
        pltpu.sync_copy(outv, dega.at[pl.ds(s * DPT, DPT)])

    @pl.when(c == 1)
    def _():
        pltpu.sync_copy(outv, degb.at[pl.ds(s * DPT, DPT)])


def _make_deg_kernel():
    return pl.kernel(
        _deg_body,
        out_type=(
            jax.ShapeDtypeStruct((NDEG,), jnp.float32),
            jax.ShapeDtypeStruct((NDEG,), jnp.float32),
        ),
        mesh=_mesh(),
        scratch_types=[
            pltpu.VMEM((NBD * LB,), jnp.int32),
            pltpu.VMEM((NDEG,), jnp.float32),
            pltpu.VMEM((NS, DPT), jnp.float32),
            pltpu.VMEM((DPT,), jnp.float32),
            pltpu.VMEM_SHARED((NS, NDEG), jnp.float32),
            pltpu.SemaphoreType.DMA,
        ],
        compiler_params=pltpu.CompilerParams(needs_layout_passes=False),
    )


def _edge_pipeline(hs, rowv, colv, bufs, acc, sem_g, sem_s):
    nd = len(bufs)
    for j in range(nd):
        pltpu.async_copy(hs.at[rowv.at[j]], bufs[j], sem_g)

    def body(q, carry):
        b0 = q * nd
        for j in range(nd):
            b = b0 + j
            pltpu.make_async_copy(hs.at[rowv.at[b]], bufs[j], sem_g).wait()
            pltpu.async_copy(bufs[j], acc.at[colv.at[b]], sem_s, add=True)
        for j in range(nd):
            b = b0 + j

            @pl.when(b + nd < NBC)
            def _():
                pltpu.make_async_copy(bufs[j], acc.at[colv.at[b]], sem_s).wait()
                pltpu.async_copy(hs.at[rowv.at[b + nd]], bufs[j], sem_g)

        return carry

    lax.fori_loop(0, NBC // nd, body, 0)
    for j in range(nd):
        pltpu.make_async_copy(bufs[j], acc.at[colv.at[NBC - nd + j]], sem_s).wait()


def _zero_acc(s, zeros, acc):
    pltpu.sync_copy(zeros.at[pl.ds(s * RPT, RPT)], acc.at[pl.ds(s * RPT, RPT)])


def _copy_out(c, s, acc, out_a, out_b):
    @pl.when(c == 0)
    def _():
        pltpu.sync_copy(acc.at[pl.ds(s * RPT, RPT)], out_a.at[pl.ds(s * RPT, RPT)])

    @pl.when(c == 1)
    def _():
        pltpu.sync_copy(acc.at[pl.ds(s * RPT, RPT)], out_b.at[pl.ds(s * RPT, RPT)])


def _agg_scratch():
    return [
        pltpu.VMEM((NBC, LB), jnp.int32),
        pltpu.VMEM((NBC, LB), jnp.int32),
        pltpu.VMEM((LB, D_PAD), jnp.float32),
        pltpu.VMEM((LB, D_PAD), jnp.float32),
        pltpu.VMEM((LB, D_PAD), jnp.float32),
        pltpu.VMEM((LB, D_PAD), jnp.float32),
        pltpu.VMEM_SHARED((NPAD, D_PAD), jnp.float32),
        pltpu.SemaphoreType.DMA,
        pltpu.SemaphoreType.DMA,
    ]


def _agg1_sc_body(hs_a, hs_b, rowp, colp, zeros, out_a, out_b,
                  rowv, colv, buf0, buf1, buf2, buf3, acc, sem_g, sem_s):
    c = lax.axis_index("c")
    s = lax.axis_index("s")
    bufs = (buf0, buf1, buf2, buf3)
    _zero_acc(s, zeros, acc)
    plsc.subcore_barrier()

    def run(hs):
        for ci in range(NCH1):
            pltpu.sync_copy(rowp.at[s].at[pl.ds(ci * NBC, NBC)], rowv)
            pltpu.sync_copy(colp.at[s].at[pl.ds(ci * NBC, NBC)], colv)
            _edge_pipeline(hs, rowv, colv, bufs, acc, sem_g, sem_s)

    @pl.when(c == 0)
    def _():
        run(hs_a)

    @pl.when(c == 1)
    def _():
        run(hs_b)

    plsc.subcore_barrier()
    _copy_out(c, s, acc, out_a, out_b)


def _make_agg1_kernel():
    return pl.kernel(
        _agg1_sc_body,
        out_type=(
            jax.ShapeDtypeStruct((NPAD, D_PAD), jnp.float32),
            jax.ShapeDtypeStruct((NPAD, D_PAD), jnp.float32),
        ),
        mesh=_mesh(),
        scratch_types=_agg_scratch(),
    )


def _agg2_sc_body(hs_a, hs_b, rowp, colp, zeros, out_a, out_b,
                  rowv, colv, buf0, buf1, buf2, buf3, acc, sem_g, sem_s):
    c = lax.axis_index("c")
    s = lax.axis_index("s")
    wid = s * NC + c
    bufs = (buf0, buf1, buf2, buf3)
    _zero_acc(s, zeros, acc)
    plsc.subcore_barrier()

    def run(hs):
        for ci in range(NCH2):
            pltpu.sync_copy(rowp.at[wid].at[pl.ds(ci * NBC, NBC)], rowv)
            pltpu.sync_copy(colp.at[wid].at[pl.ds(ci * NBC, NBC)], colv)
            _edge_pipeline(hs, rowv, colv, bufs, acc, sem_g, sem_s)

    @pl.when(c == 0)
    def _():
        run(hs_a)

    @pl.when(c == 1)
    def _():
        run(hs_b)

    plsc.subcore_barrier()
    _copy_out(c, s, acc, out_a, out_b)


def _make_agg2_kernel():
    return pl.kernel(
        _agg2_sc_body,
        out_type=(
            jax.ShapeDtypeStruct((NPAD, D_OUT2), jnp.float32),
            jax.ShapeDtypeStruct((NPAD, D_OUT2), jnp.float32),
        ),
        mesh=_mesh(),
        scratch_types=[
            pltpu.VMEM((NBC, LB), jnp.int32),
            pltpu.VMEM((NBC, LB), jnp.int32),
            pltpu.VMEM((LB, D_OUT2), jnp.float32),
            pltpu.VMEM((LB, D_OUT2), jnp.float32),
            pltpu.VMEM((LB, D_OUT2), jnp.float32),
            pltpu.VMEM((LB, D_OUT2), jnp.float32),
            pltpu.VMEM_SHARED((NPAD, D_OUT2), jnp.float32),
            pltpu.SemaphoreType.DMA,
            pltpu.SemaphoreType.DMA,
        ],
        compiler_params=pltpu.CompilerParams(use_tc_tiling_on_sc=False),
    )


def _dinv_of(dega, degb):
    deg = dega + degb + 1.0
    return lax.rsqrt(deg)


def _mmprep_body(x_ref, w_ref, dega_ref, degb_ref,
                 hs1a_ref, hs1b_ref, hss1_ref):
    h1 = jnp.dot(x_ref[...], w_ref[...], preferred_element_type=jnp.float32)
    dinv = _dinv_of(dega_ref[...], degb_ref[...])
    hs = h1 * dinv
    hs1a_ref[...] = hs[:, :D_PAD]
    hs1b_ref[...] = hs[:, D_PAD:]
    hss1_ref[...] = hs * dinv


def _tc_mmprep(x, w1, dega, degb):
    return pl.pallas_call(
        _mmprep_body,
        grid=(GRID,),
        in_specs=[
            pl.BlockSpec((BLK, D_IN), lambda i: (i, 0)),
            pl.BlockSpec((D_IN, D_HID), lambda i: (0, 0)),
            pl.BlockSpec((BLK, 1), lambda i: (i, 0)),
            pl.BlockSpec((BLK, 1), lambda i: (i, 0)),
        ],
        out_specs=(
            pl.BlockSpec((BLK, D_PAD), lambda i: (i, 0)),
            pl.BlockSpec((BLK, D_PAD), lambda i: (i, 0)),
            pl.BlockSpec((BLK, D_HID), lambda i: (i, 0)),
        ),
        out_shape=(
            jax.ShapeDtypeStruct((N_NODES, D_PAD), jnp.float32),
            jax.ShapeDtypeStruct((N_NODES, D_PAD), jnp.float32),
            jax.ShapeDtypeStruct((N_NODES, D_HID), jnp.float32),
        ),
    )(x, w1, dega, degb)


def _mid_body(s1a_ref, s1b_ref, hss1_ref, dega_ref, degb_ref, b1_ref,
              gamma_ref, beta_ref, w2_ref,
              hs2a_ref, hs2b_ref, hss2_ref, agg_ref, sums_ref, sumsq_ref):
    p = pl.program_id(0)
    j = pl.program_id(1)
    dinv = _dinv_of(dega_ref[...], degb_ref[...])

    @pl.when(p == 0)
    def _():
        s = jnp.concatenate([s1a_ref[...], s1b_ref[...]], axis=1)
        agg = dinv * s + hss1_ref[...] + b1_ref[...]
        agg_ref[pl.ds(j * BLK, BLK), :] = agg

        @pl.when(j == 0)
        def _():
            sums_ref[...] = jnp.zeros_like(sums_ref)
            sumsq_ref[...] = jnp.zeros_like(sumsq_ref)

        sums_ref[...] += jnp.sum(agg, axis=0, keepdims=True)
        sumsq_ref[...] += jnp.sum(agg * agg, axis=0, keepdims=True)

    @pl.when(p == 1)
    def _():
        inv_n = 1.0 / N_NODES
        mu = sums_ref[...] * inv_n
        var = sumsq_ref[...] * inv_n - mu * mu
        scale = gamma_ref[...] * lax.rsqrt(var + 1e-5)
        hn = (agg_ref[pl.ds(j * BLK, BLK), :] - mu) * scale + beta_ref[...]
        hn = jnp.maximum(hn, 0.0)
        h2 = jnp.dot(hn, w2_ref[...], preferred_element_type=jnp.float32)
        hs2 = h2 * dinv
        hs2a_ref[...] = hs2
        hs2b_ref[...] = hs2
        hss2_ref[...] = hs2 * dinv


def _tc_mid(s1a, s1b, hss1, dega, degb, b1, gamma1, beta1, w2p):
    def rowmap(p, j):
        return (j, 0)

    def constmap(p, j):
        return (0, 0)

    def outmap(p, j):
        return (j * p, 0)

    return pl.pallas_call(
        _mid_body,
        grid=(2, GRID),
        in_specs=[
            pl.BlockSpec((BLK, D_PAD), rowmap),
            pl.BlockSpec((BLK, D_PAD), rowmap),
            pl.BlockSpec((BLK, D_HID), rowmap),
            pl.BlockSpec((BLK, 1), rowmap),
            pl.BlockSpec((BLK, 1), rowmap),
            pl.BlockSpec((1, D_HID), constmap),
            pl.BlockSpec((1, D_HID), constmap),
            pl.BlockSpec((1, D_HID), constmap),
            pl.BlockSpec((D_HID, D_OUT2), constmap),
        ],
        out_specs=(
            pl.BlockSpec((BLK, D_OUT2), outmap),
            pl.BlockSpec((BLK, D_OUT2), outmap),
            pl.BlockSpec((BLK, D_OUT2), outmap),
        ),
        out_shape=(
            jax.ShapeDtypeStruct((N_NODES, D_OUT2), jnp.float32),
            jax.ShapeDtypeStruct((N_NODES, D_OUT2), jnp.float32),
            jax.ShapeDtypeStruct((N_NODES, D_OUT2), jnp.float32),
        ),
        scratch_shapes=[
            pltpu.VMEM((N_NODES, D_HID), jnp.float32),
            pltpu.VMEM((1, D_HID), jnp.float32),
            pltpu.VMEM((1, D_HID), jnp.float32),
        ],
    )(s1a, s1b, hss1, dega, degb, b1, gamma1, beta1, w2p)


def _fin_body(s2a_ref, s2b_ref, hss2_ref, dega_ref, degb_ref, b2_ref, o_ref):
    dinv = _dinv_of(dega_ref[...], degb_ref[...])
    s = s2a_ref[...] + s2b_ref[...]
    o_ref[...] = (dinv * s + hss2_ref[...] + b2_ref[...])[:, :D_OUT]


def _tc_final(s2a, s2b, hss2, dega, degb, b2p):
    return pl.pallas_call(
        _fin_body,
        grid=(GRID,),
        in_specs=[
            pl.BlockSpec((BLK, D_OUT2), lambda i: (i, 0)),
            pl.BlockSpec((BLK, D_OUT2), lambda i: (i, 0)),
            pl.BlockSpec((BLK, D_OUT2), lambda i: (i, 0)),
            pl.BlockSpec((BLK, 1), lambda i: (i, 0)),
            pl.BlockSpec((BLK, 1), lambda i: (i, 0)),
            pl.BlockSpec((1, D_OUT2), lambda i: (0, 0)),
        ],
        out_specs=pl.BlockSpec((BLK, D_OUT), lambda i: (i, 0)),
        out_shape=jax.ShapeDtypeStruct((N_NODES, D_OUT), jnp.float32),
    )(s2a, s2b, hss2, dega, degb, b2p)


def kernel(x, edge_index, W1, b1, gamma1, beta1, W2, b2):
    row = edge_index[0].astype(jnp.int32)
    col = edge_index[1].astype(jnp.int32)

    pad1 = NB1 * LB - EPT
    rowp = jnp.pad(row.reshape(NS, EPT), ((0, 0), (0, pad1))).reshape(NS, NB1, LB)
    colp = jnp.pad(col.reshape(NS, EPT), ((0, 0), (0, pad1)),
                   constant_values=N_NODES).reshape(NS, NB1, LB)
    padd = NB2 * LB - EPW
    rowd = jnp.pad(row.reshape(NS * NC, EPW), ((0, 0), (0, padd))
                   ).reshape(NS * NC, NB2, LB)
    cold = jnp.pad(col.reshape(NS * NC, EPW), ((0, 0), (0, padd)),
                   constant_values=N_NODES).reshape(NS * NC, NB2, LB)
    cold_flat = cold.reshape(NS * NC, NB2 * LB)

    zeros128 = jnp.zeros((NPAD, D_PAD), jnp.float32)
    zeros64 = jnp.zeros((NPAD, D_OUT2), jnp.float32)

    b1r = b1.reshape(1, D_HID)
    g1r = gamma1.reshape(1, D_HID)
    be1r = beta1.reshape(1, D_HID)
    w2p = jnp.pad(W2, ((0, 0), (0, D_OUT2 - D_OUT)))
    b2p = jnp.pad(b2, (0, D_OUT2 - D_OUT)).reshape(1, D_OUT2)

    dega_p, degb_p = _make_deg_kernel()(cold_flat)
    dega = dega_p.reshape(NDEG, 1)
    degb = degb_p.reshape(NDEG, 1)

    hs1a, hs1b, hss1 = _tc_mmprep(x, W1, dega, degb)
    s1a_p, s1b_p = _make_agg1_kernel()(hs1a, hs1b, rowp, colp, zeros128)
    hs2a, hs2b, hss2 = _tc_mid(s1a_p, s1b_p, hss1, dega, degb,
                               b1r, g1r, be1r, w2p)
    s2a_p, s2b_p = _make_agg2_kernel()(hs2a, hs2b, rowd, cold, zeros64)
    return _tc_final(s2a_p, s2b_p, hss2, dega, degb, b2p)


# speedup vs baseline: 11.9661x; 1.0196x over previous
import jax
import jax.numpy as jnp
from jax import lax
from jax.experimental import pallas as pl
from jax.experimental.pallas import tpu as pltpu
from jax.experimental.pallas import tpu_sc as plsc

N_NODES = 10000
N_EDGES = 160000
D_IN = 256
D_HID = 256
D_OUT = 40
D_PAD = 128
D_OUT2 = 48

NS = 16
NC = 2
LB = 64
NBC = 40
EPT = N_EDGES // NS
NB1 = 160
NCH1 = NB1 // NBC
EPW = N_EDGES // (NS * NC)
NB2 = 80
NCH2 = NB2 // NBC
NBD = NB2
NPAD = 10112
RPT = NPAD // NS
NDEG = 10240
DPT = NDEG // NS

BLK = 1000
GRID = N_NODES // BLK


def _mesh():
    return plsc.VectorSubcoreMesh(core_axis_name="c", subcore_axis_name="s")


def _deg_body(colp, dega, degb, colv, deg_t, redv, outv, stage, sem):
    c = lax.axis_index("c")
    s = lax.axis_index("s")
    wid = s * NC + c
    pltpu.sync_copy(colp.at[wid], colv)
    zero16 = jnp.zeros((16,), jnp.float32)

    def zbody(k, carry):
        deg_t[pl.ds(k * 16, 16)] = zero16
        return carry

    lax.fori_loop(0, NDEG // 16, zbody, 0)

    one16 = jnp.ones((16,), jnp.float32)

    def sbody(k, carry):
        idx = colv[pl.ds(k * 16, 16)]
        plsc.addupdate_scatter(deg_t, [idx], one16)
        return carry

    lax.fori_loop(0, (NBD * LB) // 16, sbody, 0)

    pltpu.sync_copy(deg_t, stage.at[s])
    plsc.subcore_barrier()

    pltpu.sync_copy(stage.at[:, pl.ds(s * DPT, DPT)], redv)

    def rbody(j, carry):
        acc = redv[0, pl.ds(j * 16, 16)]
        for t in range(1, NS):
            acc = acc + redv[t, pl.ds(j * 16, 16)]
        outv[pl.ds(j * 16, 16)] = acc
        return carry

    lax.fori_loop(0, DPT // 16, rbody, 0)

    @pl.when(c == 0)
    def _():
        pltpu.sync_copy(outv, dega.at[pl.ds(s * DPT, DPT)])

    @pl.when(c == 1)
    def _():
        pltpu.sync_copy(outv, degb.at[pl.ds(s * DPT, DPT)])


def _make_deg_kernel():
    return pl.kernel(
        _deg_body,
        out_type=(
            jax.ShapeDtypeStruct((NDEG,), jnp.float32),
            jax.ShapeDtypeStruct((NDEG,), jnp.float32),
        ),
        mesh=_mesh(),
        scratch_types=[
            pltpu.VMEM((NBD * LB,), jnp.int32),
            pltpu.VMEM((NDEG,), jnp.float32),
            pltpu.VMEM((NS, DPT), jnp.float32),
            pltpu.VMEM((DPT,), jnp.float32),
            pltpu.VMEM_SHARED((NS, NDEG), jnp.float32),
            pltpu.SemaphoreType.DMA,
        ],
        compiler_params=pltpu.CompilerParams(needs_layout_passes=False),
    )


def _edge_pipeline(hs, rowv, colv, bufs, acc, sem_g, sem_s):
    nd = len(bufs)
    for j in range(nd):
        pltpu.async_copy(hs.at[rowv.at[j]], bufs[j], sem_g)

    def body(q, carry):
        b0 = q * nd
        for j in range(nd):
            b = b0 + j
            pltpu.make_async_copy(hs.at[rowv.at[b]], bufs[j], sem_g).wait()
            pltpu.async_copy(bufs[j], acc.at[colv.at[b]], sem_s, add=True)
        for j in range(nd):
            b = b0 + j

            @pl.when(b + nd < NBC)
            def _():
                pltpu.make_async_copy(bufs[j], acc.at[colv.at[b]], sem_s).wait()
                pltpu.async_copy(hs.at[rowv.at[b + nd]], bufs[j], sem_g)

        return carry

    lax.fori_loop(0, NBC // nd, body, 0)
    for j in range(nd):
        pltpu.make_async_copy(bufs[j], acc.at[colv.at[NBC - nd + j]], sem_s).wait()


def _zero_acc(s, zeros, acc):
    pltpu.sync_copy(zeros.at[pl.ds(s * RPT, RPT)], acc.at[pl.ds(s * RPT, RPT)])


def _copy_out(c, s, acc, out_a, out_b):
    @pl.when(c == 0)
    def _():
        pltpu.sync_copy(acc.at[pl.ds(s * RPT, RPT)], out_a.at[pl.ds(s * RPT, RPT)])

    @pl.when(c == 1)
    def _():
        pltpu.sync_copy(acc.at[pl.ds(s * RPT, RPT)], out_b.at[pl.ds(s * RPT, RPT)])


def _agg_scratch():
    return [
        pltpu.VMEM((NBC, LB), jnp.int32),
        pltpu.VMEM((NBC, LB), jnp.int32),
        pltpu.VMEM((LB, D_PAD), jnp.float32),
        pltpu.VMEM((LB, D_PAD), jnp.float32),
        pltpu.VMEM((LB, D_PAD), jnp.float32),
        pltpu.VMEM((LB, D_PAD), jnp.float32),
        pltpu.VMEM_SHARED((NPAD, D_PAD), jnp.float32),
        pltpu.SemaphoreType.DMA,
        pltpu.SemaphoreType.DMA,
    ]


def _agg1_sc_body(hs_a, hs_b, rowp, colp, zeros, out_a, out_b,
                  rowv, colv, buf0, buf1, buf2, buf3, acc, sem_g, sem_s):
    c = lax.axis_index("c")
    s = lax.axis_index("s")
    bufs = (buf0, buf1, buf2, buf3)
    _zero_acc(s, zeros, acc)
    plsc.subcore_barrier()

    def run(hs):
        for ci in range(NCH1):
            pltpu.sync_copy(rowp.at[s].at[pl.ds(ci * NBC, NBC)], rowv)
            pltpu.sync_copy(colp.at[s].at[pl.ds(ci * NBC, NBC)], colv)
            _edge_pipeline(hs, rowv, colv, bufs, acc, sem_g, sem_s)

    @pl.when(c == 0)
    def _():
        run(hs_a)

    @pl.when(c == 1)
    def _():
        run(hs_b)

    plsc.subcore_barrier()
    _copy_out(c, s, acc, out_a, out_b)


def _make_agg1_kernel():
    return pl.kernel(
        _agg1_sc_body,
        out_type=(
            jax.ShapeDtypeStruct((NPAD, D_PAD), jnp.float32),
            jax.ShapeDtypeStruct((NPAD, D_PAD), jnp.float32),
        ),
        mesh=_mesh(),
        scratch_types=_agg_scratch(),
    )


def _agg2_sc_body(hs_a, hs_b, rowp, colp, zeros, out_a, out_b,
                  rowv, colv, buf0, buf1, buf2, buf3, acc, sem_g, sem_s):
    c = lax.axis_index("c")
    s = lax.axis_index("s")
    wid = s * NC + c
    bufs = (buf0, buf1, buf2, buf3)
    _zero_acc(s, zeros, acc)
    plsc.subcore_barrier()

    def run(hs):
        for ci in range(NCH2):
            pltpu.sync_copy(rowp.at[wid].at[pl.ds(ci * NBC, NBC)], rowv)
            pltpu.sync_copy(colp.at[wid].at[pl.ds(ci * NBC, NBC)], colv)
            _edge_pipeline(hs, rowv, colv, bufs, acc, sem_g, sem_s)

    @pl.when(c == 0)
    def _():
        run(hs_a)

    @pl.when(c == 1)
    def _():
        run(hs_b)

    plsc.subcore_barrier()
    _copy_out(c, s, acc, out_a, out_b)


def _make_agg2_kernel():
    return pl.kernel(
        _agg2_sc_body,
        out_type=(
            jax.ShapeDtypeStruct((NPAD, D_OUT2), jnp.float32),
            jax.ShapeDtypeStruct((NPAD, D_OUT2), jnp.float32),
        ),
        mesh=_mesh(),
        scratch_types=[
            pltpu.VMEM((NBC, LB), jnp.int32),
            pltpu.VMEM((NBC, LB), jnp.int32),
            pltpu.VMEM((LB, D_OUT2), jnp.float32),
            pltpu.VMEM((LB, D_OUT2), jnp.float32),
            pltpu.VMEM((LB, D_OUT2), jnp.float32),
            pltpu.VMEM((LB, D_OUT2), jnp.float32),
            pltpu.VMEM_SHARED((NPAD, D_OUT2), jnp.float32),
            pltpu.SemaphoreType.DMA,
            pltpu.SemaphoreType.DMA,
        ],
        compiler_params=pltpu.CompilerParams(use_tc_tiling_on_sc=False),
    )


def _dinv_of(dega, degb):
    deg = dega + degb + 1.0
    return lax.rsqrt(deg)


def _mmprep_body(x_ref, w_ref, dega_ref, degb_ref,
                 hs1a_ref, hs1b_ref, hss1_ref):
    h1 = jnp.dot(x_ref[...], w_ref[...], preferred_element_type=jnp.float32)
    dinv = _dinv_of(dega_ref[...], degb_ref[...])
    hs = h1 * dinv
    hs1a_ref[...] = hs[:, :D_PAD]
    hs1b_ref[...] = hs[:, D_PAD:]
    hss1_ref[...] = hs * dinv


def _tc_mmprep(x, w1, dega, degb):
    return pl.pallas_call(
        _mmprep_body,
        grid=(GRID,),
        in_specs=[
            pl.BlockSpec((BLK, D_IN), lambda i: (i, 0)),
            pl.BlockSpec((D_IN, D_HID), lambda i: (0, 0)),
            pl.BlockSpec((BLK, 1), lambda i: (i, 0)),
            pl.BlockSpec((BLK, 1), lambda i: (i, 0)),
        ],
        out_specs=(
            pl.BlockSpec((BLK, D_PAD), lambda i: (i, 0)),
            pl.BlockSpec((BLK, D_PAD), lambda i: (i, 0)),
            pl.BlockSpec((BLK, D_HID), lambda i: (i, 0)),
        ),
        out_shape=(
            jax.ShapeDtypeStruct((N_NODES, D_PAD), jnp.float32),
            jax.ShapeDtypeStruct((N_NODES, D_PAD), jnp.float32),
            jax.ShapeDtypeStruct((N_NODES, D_HID), jnp.float32),
        ),
    )(x, w1, dega, degb)


def _mid_body(s1a_ref, s1b_ref, hss1_ref, dega_ref, degb_ref, b1_ref,
              gamma_ref, beta_ref, w2_ref,
              hs2a_ref, hs2b_ref, hss2_ref, agg_ref, sums_ref, sumsq_ref):
    p = pl.program_id(0)
    j = pl.program_id(1)
    dinv = _dinv_of(dega_ref[...], degb_ref[...])

    @pl.when(p == 0)
    def _():
        s = jnp.concatenate([s1a_ref[...], s1b_ref[...]], axis=1)
        agg = dinv * s + hss1_ref[...] + b1_ref[...]
        agg_ref[pl.ds(j * BLK, BLK), :] = agg

        @pl.when(j == 0)
        def _():
            sums_ref[...] = jnp.zeros_like(sums_ref)
            sumsq_ref[...] = jnp.zeros_like(sumsq_ref)

        sums_ref[...] += jnp.sum(agg, axis=0, keepdims=True)
        sumsq_ref[...] += jnp.sum(agg * agg, axis=0, keepdims=True)

    @pl.when(p == 1)
    def _():
        inv_n = 1.0 / N_NODES
        mu = sums_ref[...] * inv_n
        var = sumsq_ref[...] * inv_n - mu * mu
        scale = gamma_ref[...] * lax.rsqrt(var + 1e-5)
        hn = (agg_ref[pl.ds(j * BLK, BLK), :] - mu) * scale + beta_ref[...]
        hn = jnp.maximum(hn, 0.0)
        h2 = jnp.dot(hn, w2_ref[...], preferred_element_type=jnp.float32)
        hs2 = h2 * dinv
        hs2a_ref[...] = hs2
        hs2b_ref[...] = hs2
        hss2_ref[...] = hs2 * dinv


def _tc_mid(s1a, s1b, hss1, dega, degb, b1, gamma1, beta1, w2p):
    def rowmap(p, j):
        return (j, 0)

    def constmap(p, j):
        return (0, 0)

    def outmap(p, j):
        return (j * p, 0)

    return pl.pallas_call(
        _mid_body,
        grid=(2, GRID),
        in_specs=[
            pl.BlockSpec((BLK, D_PAD), rowmap),
            pl.BlockSpec((BLK, D_PAD), rowmap),
            pl.BlockSpec((BLK, D_HID), rowmap),
            pl.BlockSpec((BLK, 1), rowmap),
            pl.BlockSpec((BLK, 1), rowmap),
            pl.BlockSpec((1, D_HID), constmap),
            pl.BlockSpec((1, D_HID), constmap),
            pl.BlockSpec((1, D_HID), constmap),
            pl.BlockSpec((D_HID, D_OUT2), constmap),
        ],
        out_specs=(
            pl.BlockSpec((BLK, D_OUT2), outmap),
            pl.BlockSpec((BLK, D_OUT2), outmap),
            pl.BlockSpec((BLK, D_OUT2), outmap),
        ),
        out_shape=(
            jax.ShapeDtypeStruct((N_NODES, D_OUT2), jnp.float32),
            jax.ShapeDtypeStruct((N_NODES, D_OUT2), jnp.float32),
            jax.ShapeDtypeStruct((N_NODES, D_OUT2), jnp.float32),
        ),
        scratch_shapes=[
            pltpu.VMEM((N_NODES, D_HID), jnp.float32),
            pltpu.VMEM((1, D_HID), jnp.float32),
            pltpu.VMEM((1, D_HID), jnp.float32),
        ],
    )(s1a, s1b, hss1, dega, degb, b1, gamma1, beta1, w2p)


def _fin_body(s2a_ref, s2b_ref, hss2_ref, dega_ref, degb_ref, b2_ref, o_ref):
    dinv = _dinv_of(dega_ref[...], degb_ref[...])
    s = s2a_ref[...] + s2b_ref[...]
    o_ref[...] = (dinv * s + hss2_ref[...] + b2_ref[...])[:, :D_OUT]


def _tc_final(s2a, s2b, hss2, dega, degb, b2p):
    return pl.pallas_call(
        _fin_body,
        grid=(GRID,),
        in_specs=[
            pl.BlockSpec((BLK, D_OUT2), lambda i: (i, 0)),
            pl.BlockSpec((BLK, D_OUT2), lambda i: (i, 0)),
            pl.BlockSpec((BLK, D_OUT2), lambda i: (i, 0)),
            pl.BlockSpec((BLK, 1), lambda i: (i, 0)),
            pl.BlockSpec((BLK, 1), lambda i: (i, 0)),
            pl.BlockSpec((1, D_OUT2), lambda i: (0, 0)),
        ],
        out_specs=pl.BlockSpec((BLK, D_OUT), lambda i: (i, 0)),
        out_shape=jax.ShapeDtypeStruct((N_NODES, D_OUT), jnp.float32),
    )(s2a, s2b, hss2, dega, degb, b2p)


def kernel(x, edge_index, W1, b1, gamma1, beta1, W2, b2):
    row = edge_index[0].astype(jnp.int32)
    col = edge_index[1].astype(jnp.int32)

    pad1 = NB1 * LB - EPT
    rowp = jnp.pad(row.reshape(NS, EPT), ((0, 0), (0, pad1))).reshape(NS, NB1, LB)
    colp = jnp.pad(col.reshape(NS, EPT), ((0, 0), (0, pad1)),
                   constant_values=N_NODES).reshape(NS, NB1, LB)
    padd = NB2 * LB - EPW
    rowd = jnp.pad(row.reshape(NS * NC, EPW), ((0, 0), (0, padd))
                   ).reshape(NS * NC, NB2, LB)
    cold = jnp.pad(col.reshape(NS * NC, EPW), ((0, 0), (0, padd)),
                   constant_values=N_NODES).reshape(NS * NC, NB2, LB)
    cold_flat = cold.reshape(NS * NC, NB2 * LB)

    zeros128 = jnp.zeros((NPAD, D_PAD), jnp.float32)
    zeros64 = jnp.zeros((NPAD, D_OUT2), jnp.float32)

    b1r = b1.reshape(1, D_HID)
    g1r = gamma1.reshape(1, D_HID)
    be1r = beta1.reshape(1, D_HID)
    w2p = jnp.pad(W2, ((0, 0), (0, D_OUT2 - D_OUT)))
    b2p = jnp.pad(b2, (0, D_OUT2 - D_OUT)).reshape(1, D_OUT2)

    dega_p, degb_p = _make_deg_kernel()(cold_flat)
    dega = dega_p.reshape(NDEG, 1)
    degb = degb_p.reshape(NDEG, 1)

    hs1a, hs1b, hss1 = _tc_mmprep(x, W1, dega, degb)
    s1a_p, s1b_p = _make_agg1_kernel()(hs1a, hs1b, rowp, colp, zeros128)
    hs2a, hs2b, hss2 = _tc_mid(s1a_p, s1b_p, hss1, dega, degb,
                               b1r, g1r, be1r, w2p)
    s2a_p, s2b_p = _make_agg2_kernel()(hs2a, hs2b, rowd, cold, zeros64)
    return _tc_final(s2a_p, s2b_p, hss2, dega, degb, b2p)


# speedup vs baseline: 11.9778x; 1.0010x over previous
import jax
import jax.numpy as jnp
from jax import lax
from jax.experimental import pallas as pl
from jax.experimental.pallas import tpu as pltpu
from jax.experimental.pallas import tpu_sc as plsc

N_NODES = 10000
N_EDGES = 160000
D_IN = 256
D_HID = 256
D_OUT = 40
D_PAD = 128
D_OUT2 = 48

NS = 16
NC = 2
LB = 64
NBC = 40
EPT = N_EDGES // NS
NB1 = 160
NCH1 = NB1 // NBC
EPW = N_EDGES // (NS * NC)
NB2 = 80
NCH2 = NB2 // NBC
NBD = NB2
NPAD = 10112
RPT = NPAD // NS
NDEG = 10240
DPT = NDEG // NS

BLK = 1000
GRID = N_NODES // BLK


def _mesh():
    return plsc.VectorSubcoreMesh(core_axis_name="c", subcore_axis_name="s")


def _deg_body(colp, dega, degb, colv, deg_t, redv, outv, stage, sem):
    c = lax.axis_index("c")
    s = lax.axis_index("s")
    wid = s * NC + c
    pltpu.sync_copy(colp.at[wid], colv)
    zero16 = jnp.zeros((16,), jnp.float32)

    def zbody(k, carry):
        deg_t[pl.ds(k * 16, 16)] = zero16
        return carry

    lax.fori_loop(0, NDEG // 16, zbody, 0)

    one16 = jnp.ones((16,), jnp.float32)

    def sbody(k, carry):
        idx = colv[pl.ds(k * 16, 16)]
        plsc.addupdate_scatter(deg_t, [idx], one16)
        return carry

    lax.fori_loop(0, (NBD * LB) // 16, sbody, 0)

    pltpu.sync_copy(deg_t, stage.at[s])
    plsc.subcore_barrier()

    pltpu.sync_copy(stage.at[:, pl.ds(s * DPT, DPT)], redv)

    def rbody(j, carry):
        acc = redv[0, pl.ds(j * 16, 16)]
        for t in range(1, NS):
            acc = acc + redv[t, pl.ds(j * 16, 16)]
        outv[pl.ds(j * 16, 16)] = acc
        return carry

    lax.fori_loop(0, DPT // 16, rbody, 0)

    @pl.when(c == 0)
    def _():
        pltpu.sync_copy(outv, dega.at[pl.ds(s * DPT, DPT)])

    @pl.when(c == 1)
    def _():
        pltpu.sync_copy(outv, degb.at[pl.ds(s * DPT, DPT)])


def _make_deg_kernel():
    return pl.kernel(
        _deg_body,
        out_type=(
            jax.ShapeDtypeStruct((NDEG,), jnp.float32),
            jax.ShapeDtypeStruct((NDEG,), jnp.float32),
        ),
        mesh=_mesh(),
        scratch_types=[
            pltpu.VMEM((NBD * LB,), jnp.int32),
            pltpu.VMEM((NDEG,), jnp.float32),
            pltpu.VMEM((NS, DPT), jnp.float32),
            pltpu.VMEM((DPT,), jnp.float32),
            pltpu.VMEM_SHARED((NS, NDEG), jnp.float32),
            pltpu.SemaphoreType.DMA,
        ],
        compiler_params=pltpu.CompilerParams(needs_layout_passes=False),
    )


def _edge_pipeline(hs, rowv, colv, bufs, acc, sem_g, sem_s):
    nd = len(bufs)
    for j in range(nd):
        pltpu.async_copy(hs.at[rowv.at[j]], bufs[j], sem_g)

    def body(q, carry):
        b0 = q * nd
        for j in range(nd):
            b = b0 + j
            pltpu.make_async_copy(hs.at[rowv.at[b]], bufs[j], sem_g).wait()
            pltpu.async_copy(bufs[j], acc.at[colv.at[b]], sem_s, add=True)
        for j in range(nd):
            b = b0 + j

            @pl.when(b + nd < NBC)
            def _():
                pltpu.make_async_copy(bufs[j], acc.at[colv.at[b]], sem_s).wait()
                pltpu.async_copy(hs.at[rowv.at[b + nd]], bufs[j], sem_g)

        return carry

    lax.fori_loop(0, NBC // nd, body, 0)
    for j in range(nd):
        pltpu.make_async_copy(bufs[j], acc.at[colv.at[NBC - nd + j]], sem_s).wait()


def _zero_acc(s, zeros, acc):
    pltpu.sync_copy(zeros.at[pl.ds(s * RPT, RPT)], acc.at[pl.ds(s * RPT, RPT)])


def _copy_out(c, s, acc, out_a, out_b):
    @pl.when(c == 0)
    def _():
        pltpu.sync_copy(acc.at[pl.ds(s * RPT, RPT)], out_a.at[pl.ds(s * RPT, RPT)])

    @pl.when(c == 1)
    def _():
        pltpu.sync_copy(acc.at[pl.ds(s * RPT, RPT)], out_b.at[pl.ds(s * RPT, RPT)])


def _agg_scratch():
    return [
        pltpu.VMEM((NBC, LB), jnp.int32),
        pltpu.VMEM((NBC, LB), jnp.int32),
        pltpu.VMEM((LB, D_PAD), jnp.float32),
        pltpu.VMEM((LB, D_PAD), jnp.float32),
        pltpu.VMEM((LB, D_PAD), jnp.float32),
        pltpu.VMEM((LB, D_PAD), jnp.float32),
        pltpu.VMEM_SHARED((NPAD, D_PAD), jnp.float32),
        pltpu.SemaphoreType.DMA,
        pltpu.SemaphoreType.DMA,
    ]


def _agg1_sc_body(hs_a, hs_b, rowp, colp, zeros, out_a, out_b,
                  rowv, colv, buf0, buf1, buf2, buf3, acc, sem_g, sem_s):
    c = lax.axis_index("c")
    s = lax.axis_index("s")
    bufs = (buf0, buf1, buf2, buf3)
    _zero_acc(s, zeros, acc)
    plsc.subcore_barrier()

    def run(hs):
        for ci in range(NCH1):
            pltpu.sync_copy(rowp.at[s].at[pl.ds(ci * NBC, NBC)], rowv)
            pltpu.sync_copy(colp.at[s].at[pl.ds(ci * NBC, NBC)], colv)
            _edge_pipeline(hs, rowv, colv, bufs, acc, sem_g, sem_s)

    @pl.when(c == 0)
    def _():
        run(hs_a)

    @pl.when(c == 1)
    def _():
        run(hs_b)

    plsc.subcore_barrier()
    _copy_out(c, s, acc, out_a, out_b)


def _make_agg1_kernel():
    return pl.kernel(
        _agg1_sc_body,
        out_type=(
            jax.ShapeDtypeStruct((NPAD, D_PAD), jnp.float32),
            jax.ShapeDtypeStruct((NPAD, D_PAD), jnp.float32),
        ),
        mesh=_mesh(),
        scratch_types=_agg_scratch(),
        compiler_params=pltpu.CompilerParams(use_tc_tiling_on_sc=False),
    )


def _agg2_sc_body(hs_a, hs_b, rowp, colp, zeros, out_a, out_b,
                  rowv, colv, buf0, buf1, buf2, buf3, acc, sem_g, sem_s):
    c = lax.axis_index("c")
    s = lax.axis_index("s")
    wid = s * NC + c
    bufs = (buf0, buf1, buf2, buf3)
    _zero_acc(s, zeros, acc)
    plsc.subcore_barrier()

    def run(hs):
        for ci in range(NCH2):
            pltpu.sync_copy(rowp.at[wid].at[pl.ds(ci * NBC, NBC)], rowv)
            pltpu.sync_copy(colp.at[wid].at[pl.ds(ci * NBC, NBC)], colv)
            _edge_pipeline(hs, rowv, colv, bufs, acc, sem_g, sem_s)

    @pl.when(c == 0)
    def _():
        run(hs_a)

    @pl.when(c == 1)
    def _():
        run(hs_b)

    plsc.subcore_barrier()
    _copy_out(c, s, acc, out_a, out_b)


def _make_agg2_kernel():
    return pl.kernel(
        _agg2_sc_body,
        out_type=(
            jax.ShapeDtypeStruct((NPAD, D_OUT2), jnp.float32),
            jax.ShapeDtypeStruct((NPAD, D_OUT2), jnp.float32),
        ),
        mesh=_mesh(),
        scratch_types=[
            pltpu.VMEM((NBC, LB), jnp.int32),
            pltpu.VMEM((NBC, LB), jnp.int32),
            pltpu.VMEM((LB, D_OUT2), jnp.float32),
            pltpu.VMEM((LB, D_OUT2), jnp.float32),
            pltpu.VMEM((LB, D_OUT2), jnp.float32),
            pltpu.VMEM((LB, D_OUT2), jnp.float32),
            pltpu.VMEM_SHARED((NPAD, D_OUT2), jnp.float32),
            pltpu.SemaphoreType.DMA,
            pltpu.SemaphoreType.DMA,
        ],
        compiler_params=pltpu.CompilerParams(use_tc_tiling_on_sc=False),
    )


def _dinv_of(dega, degb):
    deg = dega + degb + 1.0
    return lax.rsqrt(deg)


def _mmprep_body(x_ref, w_ref, dega_ref, degb_ref,
                 hs1a_ref, hs1b_ref, hss1_ref):
    h1 = jnp.dot(x_ref[...], w_ref[...], preferred_element_type=jnp.float32)
    dinv = _dinv_of(dega_ref[...], degb_ref[...])
    hs = h1 * dinv
    hs1a_ref[...] = hs[:, :D_PAD]
    hs1b_ref[...] = hs[:, D_PAD:]
    hss1_ref[...] = hs * dinv


def _tc_mmprep(x, w1, dega, degb):
    return pl.pallas_call(
        _mmprep_body,
        grid=(GRID,),
        in_specs=[
            pl.BlockSpec((BLK, D_IN), lambda i: (i, 0)),
            pl.BlockSpec((D_IN, D_HID), lambda i: (0, 0)),
            pl.BlockSpec((BLK, 1), lambda i: (i, 0)),
            pl.BlockSpec((BLK, 1), lambda i: (i, 0)),
        ],
        out_specs=(
            pl.BlockSpec((BLK, D_PAD), lambda i: (i, 0)),
            pl.BlockSpec((BLK, D_PAD), lambda i: (i, 0)),
            pl.BlockSpec((BLK, D_HID), lambda i: (i, 0)),
        ),
        out_shape=(
            jax.ShapeDtypeStruct((N_NODES, D_PAD), jnp.float32),
            jax.ShapeDtypeStruct((N_NODES, D_PAD), jnp.float32),
            jax.ShapeDtypeStruct((N_NODES, D_HID), jnp.float32),
        ),
    )(x, w1, dega, degb)


def _mid_body(s1a_ref, s1b_ref, hss1_ref, dega_ref, degb_ref, b1_ref,
              gamma_ref, beta_ref, w2_ref,
              hs2a_ref, hs2b_ref, hss2_ref, agg_ref, sums_ref, sumsq_ref):
    p = pl.program_id(0)
    j = pl.program_id(1)
    dinv = _dinv_of(dega_ref[...], degb_ref[...])

    @pl.when(p == 0)
    def _():
        s = jnp.concatenate([s1a_ref[...], s1b_ref[...]], axis=1)
        agg = dinv * s + hss1_ref[...] + b1_ref[...]
        agg_ref[pl.ds(j * BLK, BLK), :] = agg

        @pl.when(j == 0)
        def _():
            sums_ref[...] = jnp.zeros_like(sums_ref)
            sumsq_ref[...] = jnp.zeros_like(sumsq_ref)

        sums_ref[...] += jnp.sum(agg, axis=0, keepdims=True)
        sumsq_ref[...] += jnp.sum(agg * agg, axis=0, keepdims=True)

    @pl.when(p == 1)
    def _():
        inv_n = 1.0 / N_NODES
        mu = sums_ref[...] * inv_n
        var = sumsq_ref[...] * inv_n - mu * mu
        scale = gamma_ref[...] * lax.rsqrt(var + 1e-5)
        hn = (agg_ref[pl.ds(j * BLK, BLK), :] - mu) * scale + beta_ref[...]
        hn = jnp.maximum(hn, 0.0)
        h2 = jnp.dot(hn, w2_ref[...], preferred_element_type=jnp.float32)
        hs2 = h2 * dinv
        hs2a_ref[...] = hs2
        hs2b_ref[...] = hs2
        hss2_ref[...] = hs2 * dinv


def _tc_mid(s1a, s1b, hss1, dega, degb, b1, gamma1, beta1, w2p):
    def rowmap(p, j):
        return (j, 0)

    def constmap(p, j):
        return (0, 0)

    def outmap(p, j):
        return (j * p, 0)

    return pl.pallas_call(
        _mid_body,
        grid=(2, GRID),
        in_specs=[
            pl.BlockSpec((BLK, D_PAD), rowmap),
            pl.BlockSpec((BLK, D_PAD), rowmap),
            pl.BlockSpec((BLK, D_HID), rowmap),
            pl.BlockSpec((BLK, 1), rowmap),
            pl.BlockSpec((BLK, 1), rowmap),
            pl.BlockSpec((1, D_HID), constmap),
            pl.BlockSpec((1, D_HID), constmap),
            pl.BlockSpec((1, D_HID), constmap),
            pl.BlockSpec((D_HID, D_OUT2), constmap),
        ],
        out_specs=(
            pl.BlockSpec((BLK, D_OUT2), outmap),
            pl.BlockSpec((BLK, D_OUT2), outmap),
            pl.BlockSpec((BLK, D_OUT2), outmap),
        ),
        out_shape=(
            jax.ShapeDtypeStruct((N_NODES, D_OUT2), jnp.float32),
            jax.ShapeDtypeStruct((N_NODES, D_OUT2), jnp.float32),
            jax.ShapeDtypeStruct((N_NODES, D_OUT2), jnp.float32),
        ),
        scratch_shapes=[
            pltpu.VMEM((N_NODES, D_HID), jnp.float32),
            pltpu.VMEM((1, D_HID), jnp.float32),
            pltpu.VMEM((1, D_HID), jnp.float32),
        ],
    )(s1a, s1b, hss1, dega, degb, b1, gamma1, beta1, w2p)


def _fin_body(s2a_ref, s2b_ref, hss2_ref, dega_ref, degb_ref, b2_ref, o_ref):
    dinv = _dinv_of(dega_ref[...], degb_ref[...])
    s = s2a_ref[...] + s2b_ref[...]
    o_ref[...] = (dinv * s + hss2_ref[...] + b2_ref[...])[:, :D_OUT]


def _tc_final(s2a, s2b, hss2, dega, degb, b2p):
    return pl.pallas_call(
        _fin_body,
        grid=(GRID,),
        in_specs=[
            pl.BlockSpec((BLK, D_OUT2), lambda i: (i, 0)),
            pl.BlockSpec((BLK, D_OUT2), lambda i: (i, 0)),
            pl.BlockSpec((BLK, D_OUT2), lambda i: (i, 0)),
            pl.BlockSpec((BLK, 1), lambda i: (i, 0)),
            pl.BlockSpec((BLK, 1), lambda i: (i, 0)),
            pl.BlockSpec((1, D_OUT2), lambda i: (0, 0)),
        ],
        out_specs=pl.BlockSpec((BLK, D_OUT), lambda i: (i, 0)),
        out_shape=jax.ShapeDtypeStruct((N_NODES, D_OUT), jnp.float32),
    )(s2a, s2b, hss2, dega, degb, b2p)


def kernel(x, edge_index, W1, b1, gamma1, beta1, W2, b2):
    row = edge_index[0].astype(jnp.int32)
    col = edge_index[1].astype(jnp.int32)

    pad1 = NB1 * LB - EPT
    rowp = jnp.pad(row.reshape(NS, EPT), ((0, 0), (0, pad1))).reshape(NS, NB1, LB)
    colp = jnp.pad(col.reshape(NS, EPT), ((0, 0), (0, pad1)),
                   constant_values=N_NODES).reshape(NS, NB1, LB)
    padd = NB2 * LB - EPW
    rowd = jnp.pad(row.reshape(NS * NC, EPW), ((0, 0), (0, padd))
                   ).reshape(NS * NC, NB2, LB)
    cold = jnp.pad(col.reshape(NS * NC, EPW), ((0, 0), (0, padd)),
                   constant_values=N_NODES).reshape(NS * NC, NB2, LB)
    cold_flat = cold.reshape(NS * NC, NB2 * LB)

    zeros128 = jnp.zeros((NPAD, D_PAD), jnp.float32)
    zeros64 = jnp.zeros((NPAD, D_OUT2), jnp.float32)

    b1r = b1.reshape(1, D_HID)
    g1r = gamma1.reshape(1, D_HID)
    be1r = beta1.reshape(1, D_HID)
    w2p = jnp.pad(W2, ((0, 0), (0, D_OUT2 - D_OUT)))
    b2p = jnp.pad(b2, (0, D_OUT2 - D_OUT)).reshape(1, D_OUT2)

    dega_p, degb_p = _make_deg_kernel()(cold_flat)
    dega = dega_p.reshape(NDEG, 1)
    degb = degb_p.reshape(NDEG, 1)

    hs1a, hs1b, hss1 = _tc_mmprep(x, W1, dega, degb)
    s1a_p, s1b_p = _make_agg1_kernel()(hs1a, hs1b, rowp, colp, zeros128)
    hs2a, hs2b, hss2 = _tc_mid(s1a_p, s1b_p, hss1, dega, degb,
                               b1r, g1r, be1r, w2p)
    s2a_p, s2b_p = _make_agg2_kernel()(hs2a, hs2b, rowd, cold, zeros64)
    return _tc_final(s2a_p, s2b_p, hss2, dega, degb, b2p)


# speedup vs baseline: 12.2751x; 1.0248x over previous
import jax
import jax.numpy as jnp
from jax import lax
from jax.experimental import pallas as pl
from jax.experimental.pallas import tpu as pltpu
from jax.experimental.pallas import tpu_sc as plsc

N_NODES = 10000
N_EDGES = 160000
D_IN = 256
D_HID = 256
D_OUT = 40
D_PAD = 128
D_OUT2 = 48

NS = 16
NC = 2
LB = 32
ND = 8
NBC = 80
EPT = N_EDGES // NS
NB1 = 320
NCH1 = NB1 // NBC
EPW = N_EDGES // (NS * NC)
NB2 = 160
NCH2 = NB2 // NBC
EPWP = 5120
NPAD = 10112
RPT = NPAD // NS
NDEG = 10240
DPT = NDEG // NS

BLK = 1000
GRID = N_NODES // BLK


def _mesh():
    return plsc.VectorSubcoreMesh(core_axis_name="c", subcore_axis_name="s")


def _deg_body(colp, dega, degb, colv, deg_t, redv, outv, stage, sem):
    c = lax.axis_index("c")
    s = lax.axis_index("s")
    wid = s * NC + c
    pltpu.sync_copy(colp.at[wid], colv)
    zero16 = jnp.zeros((16,), jnp.float32)

    def zbody(k, carry):
        deg_t[pl.ds(k * 16, 16)] = zero16
        return carry

    lax.fori_loop(0, NDEG // 16, zbody, 0)

    one16 = jnp.ones((16,), jnp.float32)

    def sbody(k, carry):
        idx = colv[pl.ds(k * 16, 16)]
        plsc.addupdate_scatter(deg_t, [idx], one16)
        return carry

    lax.fori_loop(0, EPWP // 16, sbody, 0)

    pltpu.sync_copy(deg_t, stage.at[s])
    plsc.subcore_barrier()

    pltpu.sync_copy(stage.at[:, pl.ds(s * DPT, DPT)], redv)

    def rbody(j, carry):
        acc = redv[0, pl.ds(j * 16, 16)]
        for t in range(1, NS):
            acc = acc + redv[t, pl.ds(j * 16, 16)]
        outv[pl.ds(j * 16, 16)] = acc
        return carry

    lax.fori_loop(0, DPT // 16, rbody, 0)

    @pl.when(c == 0)
    def _():
        pltpu.sync_copy(outv, dega.at[pl.ds(s * DPT, DPT)])

    @pl.when(c == 1)
    def _():
        pltpu.sync_copy(outv, degb.at[pl.ds(s * DPT, DPT)])


def _make_deg_kernel():
    return pl.kernel(
        _deg_body,
        out_type=(
            jax.ShapeDtypeStruct((NDEG,), jnp.float32),
            jax.ShapeDtypeStruct((NDEG,), jnp.float32),
        ),
        mesh=_mesh(),
        scratch_types=[
            pltpu.VMEM((EPWP,), jnp.int32),
            pltpu.VMEM((NDEG,), jnp.float32),
            pltpu.VMEM((NS, DPT), jnp.float32),
            pltpu.VMEM((DPT,), jnp.float32),
            pltpu.VMEM_SHARED((NS, NDEG), jnp.float32),
            pltpu.SemaphoreType.DMA,
        ],
        compiler_params=pltpu.CompilerParams(needs_layout_passes=False),
    )


def _edge_pipeline(hs, rowv, colv, bufs, acc, sem_g, sem_s):
    nd = len(bufs)
    lead = 5
    lag = nd - lead
    for b in range(lead):
        pltpu.async_copy(hs.at[rowv.at[b]], bufs[b], sem_g)

    def body(q, carry):
        b0 = q * nd
        for j in range(nd):
            b = b0 + j
            jn = (j + lead) % nd
            pltpu.make_async_copy(hs.at[rowv.at[b]], bufs[j], sem_g).wait()
            pltpu.async_copy(bufs[j], acc.at[colv.at[b]], sem_s, add=True)

            @pl.when(b + lead < NBC)
            def _():
                @pl.when(b >= lag)
                def _():
                    pltpu.make_async_copy(
                        bufs[jn], acc.at[colv.at[b - lag]], sem_s).wait()

                pltpu.async_copy(hs.at[rowv.at[b + lead]], bufs[jn], sem_g)

        return carry

    lax.fori_loop(0, NBC // nd, body, 0)
    for j in range(nd):
        pltpu.make_async_copy(bufs[j], acc.at[colv.at[NBC - nd + j]], sem_s).wait()


def _zero_acc(s, zeros, acc):
    pltpu.sync_copy(zeros.at[pl.ds(s * RPT, RPT)], acc.at[pl.ds(s * RPT, RPT)])


def _copy_out(c, s, acc, out_a, out_b):
    @pl.when(c == 0)
    def _():
        pltpu.sync_copy(acc.at[pl.ds(s * RPT, RPT)], out_a.at[pl.ds(s * RPT, RPT)])

    @pl.when(c == 1)
    def _():
        pltpu.sync_copy(acc.at[pl.ds(s * RPT, RPT)], out_b.at[pl.ds(s * RPT, RPT)])


def _agg_scratch(width):
    return ([
        pltpu.VMEM((NBC, LB), jnp.int32),
        pltpu.VMEM((NBC, LB), jnp.int32)]
        + [pltpu.VMEM((LB, width), jnp.float32) for _ in range(ND)]
        + [pltpu.VMEM_SHARED((NPAD, width), jnp.float32),
           pltpu.SemaphoreType.DMA,
           pltpu.SemaphoreType.DMA])


def _agg1_sc_body(hs_a, hs_b, rowp, colp, zeros, out_a, out_b,
                  rowv, colv, *rest):
    bufs = rest[:ND]
    acc, sem_g, sem_s = rest[ND:]
    c = lax.axis_index("c")
    s = lax.axis_index("s")
    _zero_acc(s, zeros, acc)
    plsc.subcore_barrier()

    def run(hs):
        for ci in range(NCH1):
            pltpu.sync_copy(rowp.at[s].at[pl.ds(ci * NBC, NBC)], rowv)
            pltpu.sync_copy(colp.at[s].at[pl.ds(ci * NBC, NBC)], colv)
            _edge_pipeline(hs, rowv, colv, bufs, acc, sem_g, sem_s)

    @pl.when(c == 0)
    def _():
        run(hs_a)

    @pl.when(c == 1)
    def _():
        run(hs_b)

    plsc.subcore_barrier()
    _copy_out(c, s, acc, out_a, out_b)


def _make_agg1_kernel():
    return pl.kernel(
        _agg1_sc_body,
        out_type=(
            jax.ShapeDtypeStruct((NPAD, D_PAD), jnp.float32),
            jax.ShapeDtypeStruct((NPAD, D_PAD), jnp.float32),
        ),
        mesh=_mesh(),
        scratch_types=_agg_scratch(D_PAD),
        compiler_params=pltpu.CompilerParams(use_tc_tiling_on_sc=False),
    )


def _agg2_sc_body(hs_a, hs_b, rowp, colp, zeros, out_a, out_b,
                  rowv, colv, *rest):
    bufs = rest[:ND]
    acc, sem_g, sem_s = rest[ND:]
    c = lax.axis_index("c")
    s = lax.axis_index("s")
    wid = s * NC + c
    _zero_acc(s, zeros, acc)
    plsc.subcore_barrier()

    def run(hs):
        for ci in range(NCH2):
            pltpu.sync_copy(rowp.at[wid].at[pl.ds(ci * NBC, NBC)], rowv)
            pltpu.sync_copy(colp.at[wid].at[pl.ds(ci * NBC, NBC)], colv)
            _edge_pipeline(hs, rowv, colv, bufs, acc, sem_g, sem_s)

    @pl.when(c == 0)
    def _():
        run(hs_a)

    @pl.when(c == 1)
    def _():
        run(hs_b)

    plsc.subcore_barrier()
    _copy_out(c, s, acc, out_a, out_b)


def _make_agg2_kernel():
    return pl.kernel(
        _agg2_sc_body,
        out_type=(
            jax.ShapeDtypeStruct((NPAD, D_OUT2), jnp.float32),
            jax.ShapeDtypeStruct((NPAD, D_OUT2), jnp.float32),
        ),
        mesh=_mesh(),
        scratch_types=_agg_scratch(D_OUT2),
        compiler_params=pltpu.CompilerParams(use_tc_tiling_on_sc=False),
    )


def _dinv_of(dega, degb):
    deg = dega + degb + 1.0
    return lax.rsqrt(deg)


def _mmprep_body(x_ref, w_ref, dega_ref, degb_ref,
                 hs1a_ref, hs1b_ref, hss1_ref):
    h1 = jnp.dot(x_ref[...], w_ref[...], preferred_element_type=jnp.float32)
    dinv = _dinv_of(dega_ref[...], degb_ref[...])
    hs = h1 * dinv
    hs1a_ref[...] = hs[:, :D_PAD]
    hs1b_ref[...] = hs[:, D_PAD:]
    hss1_ref[...] = hs * dinv


def _tc_mmprep(x, w1, dega, degb):
    return pl.pallas_call(
        _mmprep_body,
        grid=(GRID,),
        in_specs=[
            pl.BlockSpec((BLK, D_IN), lambda i: (i, 0)),
            pl.BlockSpec((D_IN, D_HID), lambda i: (0, 0)),
            pl.BlockSpec((BLK, 1), lambda i: (i, 0)),
            pl.BlockSpec((BLK, 1), lambda i: (i, 0)),
        ],
        out_specs=(
            pl.BlockSpec((BLK, D_PAD), lambda i: (i, 0)),
            pl.BlockSpec((BLK, D_PAD), lambda i: (i, 0)),
            pl.BlockSpec((BLK, D_HID), lambda i: (i, 0)),
        ),
        out_shape=(
            jax.ShapeDtypeStruct((N_NODES, D_PAD), jnp.float32),
            jax.ShapeDtypeStruct((N_NODES, D_PAD), jnp.float32),
            jax.ShapeDtypeStruct((N_NODES, D_HID), jnp.float32),
        ),
    )(x, w1, dega, degb)


def _mid_body(s1a_ref, s1b_ref, hss1_ref, dega_ref, degb_ref, b1_ref,
              gamma_ref, beta_ref, w2_ref,
              hs2a_ref, hs2b_ref, hss2_ref, agg_ref, sums_ref, sumsq_ref):
    p = pl.program_id(0)
    j = pl.program_id(1)
    dinv = _dinv_of(dega_ref[...], degb_ref[...])

    @pl.when(p == 0)
    def _():
        s = jnp.concatenate([s1a_ref[...], s1b_ref[...]], axis=1)
        agg = dinv * s + hss1_ref[...] + b1_ref[...]
        agg_ref[pl.ds(j * BLK, BLK), :] = agg

        @pl.when(j == 0)
        def _():
            sums_ref[...] = jnp.zeros_like(sums_ref)
            sumsq_ref[...] = jnp.zeros_like(sumsq_ref)

        sums_ref[...] += jnp.sum(agg, axis=0, keepdims=True)
        sumsq_ref[...] += jnp.sum(agg * agg, axis=0, keepdims=True)

    @pl.when(p == 1)
    def _():
        inv_n = 1.0 / N_NODES
        mu = sums_ref[...] * inv_n
        var = sumsq_ref[...] * inv_n - mu * mu
        scale = gamma_ref[...] * lax.rsqrt(var + 1e-5)
        hn = (agg_ref[pl.ds(j * BLK, BLK), :] - mu) * scale + beta_ref[...]
        hn = jnp.maximum(hn, 0.0)
        h2 = jnp.dot(hn, w2_ref[...], preferred_element_type=jnp.float32)
        hs2 = h2 * dinv
        hs2a_ref[...] = hs2
        hs2b_ref[...] = hs2
        hss2_ref[...] = hs2 * dinv


def _tc_mid(s1a, s1b, hss1, dega, degb, b1, gamma1, beta1, w2p):
    def rowmap(p, j):
        return (j, 0)

    def constmap(p, j):
        return (0, 0)

    def outmap(p, j):
        return (j * p, 0)

    return pl.pallas_call(
        _mid_body,
        grid=(2, GRID),
        in_specs=[
            pl.BlockSpec((BLK, D_PAD), rowmap),
            pl.BlockSpec((BLK, D_PAD), rowmap),
            pl.BlockSpec((BLK, D_HID), rowmap),
            pl.BlockSpec((BLK, 1), rowmap),
            pl.BlockSpec((BLK, 1), rowmap),
            pl.BlockSpec((1, D_HID), constmap),
            pl.BlockSpec((1, D_HID), constmap),
            pl.BlockSpec((1, D_HID), constmap),
            pl.BlockSpec((D_HID, D_OUT2), constmap),
        ],
        out_specs=(
            pl.BlockSpec((BLK, D_OUT2), outmap),
            pl.BlockSpec((BLK, D_OUT2), outmap),
            pl.BlockSpec((BLK, D_OUT2), outmap),
        ),
        out_shape=(
            jax.ShapeDtypeStruct((N_NODES, D_OUT2), jnp.float32),
            jax.ShapeDtypeStruct((N_NODES, D_OUT2), jnp.float32),
            jax.ShapeDtypeStruct((N_NODES, D_OUT2), jnp.float32),
        ),
        scratch_shapes=[
            pltpu.VMEM((N_NODES, D_HID), jnp.float32),
            pltpu.VMEM((1, D_HID), jnp.float32),
            pltpu.VMEM((1, D_HID), jnp.float32),
        ],
    )(s1a, s1b, hss1, dega, degb, b1, gamma1, beta1, w2p)


def _fin_body(s2a_ref, s2b_ref, hss2_ref, dega_ref, degb_ref, b2_ref, o_ref):
    dinv = _dinv_of(dega_ref[...], degb_ref[...])
    s = s2a_ref[...] + s2b_ref[...]
    o_ref[...] = (dinv * s + hss2_ref[...] + b2_ref[...])[:, :D_OUT]


def _tc_final(s2a, s2b, hss2, dega, degb, b2p):
    return pl.pallas_call(
        _fin_body,
        grid=(GRID,),
        in_specs=[
            pl.BlockSpec((BLK, D_OUT2), lambda i: (i, 0)),
            pl.BlockSpec((BLK, D_OUT2), lambda i: (i, 0)),
            pl.BlockSpec((BLK, D_OUT2), lambda i: (i, 0)),
            pl.BlockSpec((BLK, 1), lambda i: (i, 0)),
            pl.BlockSpec((BLK, 1), lambda i: (i, 0)),
            pl.BlockSpec((1, D_OUT2), lambda i: (0, 0)),
        ],
        out_specs=pl.BlockSpec((BLK, D_OUT), lambda i: (i, 0)),
        out_shape=jax.ShapeDtypeStruct((N_NODES, D_OUT), jnp.float32),
    )(s2a, s2b, hss2, dega, degb, b2p)


def kernel(x, edge_index, W1, b1, gamma1, beta1, W2, b2):
    row = edge_index[0].astype(jnp.int32)
    col = edge_index[1].astype(jnp.int32)

    pad1 = NB1 * LB - EPT
    rowp = jnp.pad(row.reshape(NS, EPT), ((0, 0), (0, pad1))).reshape(NS, NB1, LB)
    colp = jnp.pad(col.reshape(NS, EPT), ((0, 0), (0, pad1)),
                   constant_values=N_NODES).reshape(NS, NB1, LB)
    padd = NB2 * LB - EPW
    rowd = jnp.pad(row.reshape(NS * NC, EPW), ((0, 0), (0, padd))
                   ).reshape(NS * NC, NB2, LB)
    cold = jnp.pad(col.reshape(NS * NC, EPW), ((0, 0), (0, padd)),
                   constant_values=N_NODES).reshape(NS * NC, NB2, LB)
    cold_flat = cold.reshape(NS * NC, NB2 * LB)

    zeros128 = jnp.zeros((NPAD, D_PAD), jnp.float32)
    zeros64 = jnp.zeros((NPAD, D_OUT2), jnp.float32)

    b1r = b1.reshape(1, D_HID)
    g1r = gamma1.reshape(1, D_HID)
    be1r = beta1.reshape(1, D_HID)
    w2p = jnp.pad(W2, ((0, 0), (0, D_OUT2 - D_OUT)))
    b2p = jnp.pad(b2, (0, D_OUT2 - D_OUT)).reshape(1, D_OUT2)

    dega_p, degb_p = _make_deg_kernel()(cold_flat)
    dega = dega_p.reshape(NDEG, 1)
    degb = degb_p.reshape(NDEG, 1)

    hs1a, hs1b, hss1 = _tc_mmprep(x, W1, dega, degb)
    s1a_p, s1b_p = _make_agg1_kernel()(hs1a, hs1b, rowp, colp, zeros128)
    hs2a, hs2b, hss2 = _tc_mid(s1a_p, s1b_p, hss1, dega, degb,
                               b1r, g1r, be1r, w2p)
    s2a_p, s2b_p = _make_agg2_kernel()(hs2a, hs2b, rowd, cold, zeros64)
    return _tc_final(s2a_p, s2b_p, hss2, dega, degb, b2p)


# speedup vs baseline: 12.3776x; 1.0084x over previous
import jax
import jax.numpy as jnp
from jax import lax
from jax.experimental import pallas as pl
from jax.experimental.pallas import tpu as pltpu
from jax.experimental.pallas import tpu_sc as plsc

N_NODES = 10000
N_EDGES = 160000
D_IN = 256
D_HID = 256
D_OUT = 40
D_PAD = 128
D_OUT2 = 48

NS = 16
NC = 2
LB = 32
ND = 8
NBC = 80
EPT = N_EDGES // NS
NB1 = 320
NCH1 = NB1 // NBC
EPW = N_EDGES // (NS * NC)
NB2 = 160
NCH2 = NB2 // NBC
EPWP = 5120
NPAD = 10112
RPT = NPAD // NS
NDEG = 10240
DPT = NDEG // NS

BLK = 1000
GRID = N_NODES // BLK


def _mesh():
    return plsc.VectorSubcoreMesh(core_axis_name="c", subcore_axis_name="s")


def _deg_body(colp, dega, degb, colv, deg_t, redv, outv, stage, sem):
    c = lax.axis_index("c")
    s = lax.axis_index("s")
    wid = s * NC + c
    pltpu.sync_copy(colp.at[wid], colv)
    zero16 = jnp.zeros((16,), jnp.float32)

    def zbody(k, carry):
        deg_t[pl.ds(k * 16, 16)] = zero16
        return carry

    lax.fori_loop(0, NDEG // 16, zbody, 0)

    one16 = jnp.ones((16,), jnp.float32)

    def sbody(k, carry):
        idx = colv[pl.ds(k * 16, 16)]
        plsc.addupdate_scatter(deg_t, [idx], one16)
        return carry

    lax.fori_loop(0, EPWP // 16, sbody, 0)

    pltpu.sync_copy(deg_t, stage.at[s])
    plsc.subcore_barrier()

    pltpu.sync_copy(stage.at[:, pl.ds(s * DPT, DPT)], redv)

    def rbody(j, carry):
        acc = redv[0, pl.ds(j * 16, 16)]
        for t in range(1, NS):
            acc = acc + redv[t, pl.ds(j * 16, 16)]
        outv[pl.ds(j * 16, 16)] = acc
        return carry

    lax.fori_loop(0, DPT // 16, rbody, 0)

    @pl.when(c == 0)
    def _():
        pltpu.sync_copy(outv, dega.at[pl.ds(s * DPT, DPT)])

    @pl.when(c == 1)
    def _():
        pltpu.sync_copy(outv, degb.at[pl.ds(s * DPT, DPT)])


def _make_deg_kernel():
    return pl.kernel(
        _deg_body,
        out_type=(
            jax.ShapeDtypeStruct((NDEG,), jnp.float32),
            jax.ShapeDtypeStruct((NDEG,), jnp.float32),
        ),
        mesh=_mesh(),
        scratch_types=[
            pltpu.VMEM((EPWP,), jnp.int32),
            pltpu.VMEM((NDEG,), jnp.float32),
            pltpu.VMEM((NS, DPT), jnp.float32),
            pltpu.VMEM((DPT,), jnp.float32),
            pltpu.VMEM_SHARED((NS, NDEG), jnp.float32),
            pltpu.SemaphoreType.DMA,
        ],
        compiler_params=pltpu.CompilerParams(needs_layout_passes=False),
    )


def _edge_pipeline(hs, rowv, colv, bufs, acc, sem_g, sem_s):
    nd = len(bufs)
    lead = 6
    lag = nd - lead
    for b in range(lead):
        pltpu.async_copy(hs.at[rowv.at[b]], bufs[b], sem_g)

    def body(q, carry):
        b0 = q * nd
        for j in range(nd):
            b = b0 + j
            jn = (j + lead) % nd
            pltpu.make_async_copy(hs.at[rowv.at[b]], bufs[j], sem_g).wait()
            pltpu.async_copy(bufs[j], acc.at[colv.at[b]], sem_s, add=True)

            @pl.when(b + lead < NBC)
            def _():
                @pl.when(b >= lag)
                def _():
                    pltpu.make_async_copy(
                        bufs[jn], acc.at[colv.at[b - lag]], sem_s).wait()

                pltpu.async_copy(hs.at[rowv.at[b + lead]], bufs[jn], sem_g)

        return carry

    lax.fori_loop(0, NBC // nd, body, 0)
    for j in range(nd):
        pltpu.make_async_copy(bufs[j], acc.at[colv.at[NBC - nd + j]], sem_s).wait()


def _zero_acc(s, zeros, acc):
    pltpu.sync_copy(zeros.at[pl.ds(s * RPT, RPT)], acc.at[pl.ds(s * RPT, RPT)])


def _copy_out(c, s, acc, out_a, out_b):
    @pl.when(c == 0)
    def _():
        pltpu.sync_copy(acc.at[pl.ds(s * RPT, RPT)], out_a.at[pl.ds(s * RPT, RPT)])

    @pl.when(c == 1)
    def _():
        pltpu.sync_copy(acc.at[pl.ds(s * RPT, RPT)], out_b.at[pl.ds(s * RPT, RPT)])


def _agg_scratch(width):
    return ([
        pltpu.VMEM((NBC, LB), jnp.int32),
        pltpu.VMEM((NBC, LB), jnp.int32)]
        + [pltpu.VMEM((LB, width), jnp.float32) for _ in range(ND)]
        + [pltpu.VMEM_SHARED((NPAD, width), jnp.float32),
           pltpu.SemaphoreType.DMA,
           pltpu.SemaphoreType.DMA])


def _agg1_sc_body(hs_a, hs_b, rowp, colp, zeros, out_a, out_b,
                  rowv, colv, *rest):
    bufs = rest[:ND]
    acc, sem_g, sem_s = rest[ND:]
    c = lax.axis_index("c")
    s = lax.axis_index("s")
    _zero_acc(s, zeros, acc)
    plsc.subcore_barrier()

    def run(hs):
        for ci in range(NCH1):
            pltpu.sync_copy(rowp.at[s].at[pl.ds(ci * NBC, NBC)], rowv)
            pltpu.sync_copy(colp.at[s].at[pl.ds(ci * NBC, NBC)], colv)
            _edge_pipeline(hs, rowv, colv, bufs, acc, sem_g, sem_s)

    @pl.when(c == 0)
    def _():
        run(hs_a)

    @pl.when(c == 1)
    def _():
        run(hs_b)

    plsc.subcore_barrier()
    _copy_out(c, s, acc, out_a, out_b)


def _make_agg1_kernel():
    return pl.kernel(
        _agg1_sc_body,
        out_type=(
            jax.ShapeDtypeStruct((NPAD, D_PAD), jnp.float32),
            jax.ShapeDtypeStruct((NPAD, D_PAD), jnp.float32),
        ),
        mesh=_mesh(),
        scratch_types=_agg_scratch(D_PAD),
        compiler_params=pltpu.CompilerParams(use_tc_tiling_on_sc=False),
    )


def _agg2_sc_body(hs_a, hs_b, rowp, colp, zeros, out_a, out_b,
                  rowv, colv, *rest):
    bufs = rest[:ND]
    acc, sem_g, sem_s = rest[ND:]
    c = lax.axis_index("c")
    s = lax.axis_index("s")
    wid = s * NC + c
    _zero_acc(s, zeros, acc)
    plsc.subcore_barrier()

    def run(hs):
        for ci in range(NCH2):
            pltpu.sync_copy(rowp.at[wid].at[pl.ds(ci * NBC, NBC)], rowv)
            pltpu.sync_copy(colp.at[wid].at[pl.ds(ci * NBC, NBC)], colv)
            _edge_pipeline(hs, rowv, colv, bufs, acc, sem_g, sem_s)

    @pl.when(c == 0)
    def _():
        run(hs_a)

    @pl.when(c == 1)
    def _():
        run(hs_b)

    plsc.subcore_barrier()
    _copy_out(c, s, acc, out_a, out_b)


def _make_agg2_kernel():
    return pl.kernel(
        _agg2_sc_body,
        out_type=(
            jax.ShapeDtypeStruct((NPAD, D_OUT2), jnp.float32),
            jax.ShapeDtypeStruct((NPAD, D_OUT2), jnp.float32),
        ),
        mesh=_mesh(),
        scratch_types=_agg_scratch(D_OUT2),
        compiler_params=pltpu.CompilerParams(use_tc_tiling_on_sc=False),
    )


def _dinv_of(dega, degb):
    deg = dega + degb + 1.0
    return lax.rsqrt(deg)


def _mmprep_body(x_ref, w_ref, dega_ref, degb_ref,
                 hs1a_ref, hs1b_ref, hss1_ref):
    h1 = jnp.dot(x_ref[...], w_ref[...], preferred_element_type=jnp.float32)
    dinv = _dinv_of(dega_ref[...], degb_ref[...])
    hs = h1 * dinv
    hs1a_ref[...] = hs[:, :D_PAD]
    hs1b_ref[...] = hs[:, D_PAD:]
    hss1_ref[...] = hs * dinv


def _tc_mmprep(x, w1, dega, degb):
    return pl.pallas_call(
        _mmprep_body,
        grid=(GRID,),
        in_specs=[
            pl.BlockSpec((BLK, D_IN), lambda i: (i, 0)),
            pl.BlockSpec((D_IN, D_HID), lambda i: (0, 0)),
            pl.BlockSpec((BLK, 1), lambda i: (i, 0)),
            pl.BlockSpec((BLK, 1), lambda i: (i, 0)),
        ],
        out_specs=(
            pl.BlockSpec((BLK, D_PAD), lambda i: (i, 0)),
            pl.BlockSpec((BLK, D_PAD), lambda i: (i, 0)),
            pl.BlockSpec((BLK, D_HID), lambda i: (i, 0)),
        ),
        out_shape=(
            jax.ShapeDtypeStruct((N_NODES, D_PAD), jnp.float32),
            jax.ShapeDtypeStruct((N_NODES, D_PAD), jnp.float32),
            jax.ShapeDtypeStruct((N_NODES, D_HID), jnp.float32),
        ),
    )(x, w1, dega, degb)


def _mid_body(s1a_ref, s1b_ref, hss1_ref, dega_ref, degb_ref, b1_ref,
              gamma_ref, beta_ref, w2_ref,
              hs2a_ref, hs2b_ref, hss2_ref, agg_ref, sums_ref, sumsq_ref):
    p = pl.program_id(0)
    j = pl.program_id(1)
    dinv = _dinv_of(dega_ref[...], degb_ref[...])

    @pl.when(p == 0)
    def _():
        s = jnp.concatenate([s1a_ref[...], s1b_ref[...]], axis=1)
        agg = dinv * s + hss1_ref[...] + b1_ref[...]
        agg_ref[pl.ds(j * BLK, BLK), :] = agg

        @pl.when(j == 0)
        def _():
            sums_ref[...] = jnp.zeros_like(sums_ref)
            sumsq_ref[...] = jnp.zeros_like(sumsq_ref)

        sums_ref[...] += jnp.sum(agg, axis=0, keepdims=True)
        sumsq_ref[...] += jnp.sum(agg * agg, axis=0, keepdims=True)

    @pl.when(p == 1)
    def _():
        inv_n = 1.0 / N_NODES
        mu = sums_ref[...] * inv_n
        var = sumsq_ref[...] * inv_n - mu * mu
        scale = gamma_ref[...] * lax.rsqrt(var + 1e-5)
        hn = (agg_ref[pl.ds(j * BLK, BLK), :] - mu) * scale + beta_ref[...]
        hn = jnp.maximum(hn, 0.0)
        h2 = jnp.dot(hn, w2_ref[...], preferred_element_type=jnp.float32)
        hs2 = h2 * dinv
        hs2a_ref[...] = hs2
        hs2b_ref[...] = hs2
        hss2_ref[...] = hs2 * dinv


def _tc_mid(s1a, s1b, hss1, dega, degb, b1, gamma1, beta1, w2p):
    def rowmap(p, j):
        return (j, 0)

    def constmap(p, j):
        return (0, 0)

    def outmap(p, j):
        return (j * p, 0)

    return pl.pallas_call(
        _mid_body,
        grid=(2, GRID),
        in_specs=[
            pl.BlockSpec((BLK, D_PAD), rowmap),
            pl.BlockSpec((BLK, D_PAD), rowmap),
            pl.BlockSpec((BLK, D_HID), rowmap),
            pl.BlockSpec((BLK, 1), rowmap),
            pl.BlockSpec((BLK, 1), rowmap),
            pl.BlockSpec((1, D_HID), constmap),
            pl.BlockSpec((1, D_HID), constmap),
            pl.BlockSpec((1, D_HID), constmap),
            pl.BlockSpec((D_HID, D_OUT2), constmap),
        ],
        out_specs=(
            pl.BlockSpec((BLK, D_OUT2), outmap),
            pl.BlockSpec((BLK, D_OUT2), outmap),
            pl.BlockSpec((BLK, D_OUT2), outmap),
        ),
        out_shape=(
            jax.ShapeDtypeStruct((N_NODES, D_OUT2), jnp.float32),
            jax.ShapeDtypeStruct((N_NODES, D_OUT2), jnp.float32),
            jax.ShapeDtypeStruct((N_NODES, D_OUT2), jnp.float32),
        ),
        scratch_shapes=[
            pltpu.VMEM((N_NODES, D_HID), jnp.float32),
            pltpu.VMEM((1, D_HID), jnp.float32),
            pltpu.VMEM((1, D_HID), jnp.float32),
        ],
    )(s1a, s1b, hss1, dega, degb, b1, gamma1, beta1, w2p)


def _fin_body(s2a_ref, s2b_ref, hss2_ref, dega_ref, degb_ref, b2_ref, o_ref):
    dinv = _dinv_of(dega_ref[...], degb_ref[...])
    s = s2a_ref[...] + s2b_ref[...]
    o_ref[...] = (dinv * s + hss2_ref[...] + b2_ref[...])[:, :D_OUT]


def _tc_final(s2a, s2b, hss2, dega, degb, b2p):
    return pl.pallas_call(
        _fin_body,
        grid=(GRID,),
        in_specs=[
            pl.BlockSpec((BLK, D_OUT2), lambda i: (i, 0)),
            pl.BlockSpec((BLK, D_OUT2), lambda i: (i, 0)),
            pl.BlockSpec((BLK, D_OUT2), lambda i: (i, 0)),
            pl.BlockSpec((BLK, 1), lambda i: (i, 0)),
            pl.BlockSpec((BLK, 1), lambda i: (i, 0)),
            pl.BlockSpec((1, D_OUT2), lambda i: (0, 0)),
        ],
        out_specs=pl.BlockSpec((BLK, D_OUT), lambda i: (i, 0)),
        out_shape=jax.ShapeDtypeStruct((N_NODES, D_OUT), jnp.float32),
    )(s2a, s2b, hss2, dega, degb, b2p)


def kernel(x, edge_index, W1, b1, gamma1, beta1, W2, b2):
    row = edge_index[0].astype(jnp.int32)
    col = edge_index[1].astype(jnp.int32)

    pad1 = NB1 * LB - EPT
    rowp = jnp.pad(row.reshape(NS, EPT), ((0, 0), (0, pad1))).reshape(NS, NB1, LB)
    colp = jnp.pad(col.reshape(NS, EPT), ((0, 0), (0, pad1)),
                   constant_values=N_NODES).reshape(NS, NB1, LB)
    padd = NB2 * LB - EPW
    rowd = jnp.pad(row.reshape(NS * NC, EPW), ((0, 0), (0, padd))
                   ).reshape(NS * NC, NB2, LB)
    cold = jnp.pad(col.reshape(NS * NC, EPW), ((0, 0), (0, padd)),
                   constant_values=N_NODES).reshape(NS * NC, NB2, LB)
    cold_flat = cold.reshape(NS * NC, NB2 * LB)

    zeros128 = jnp.zeros((NPAD, D_PAD), jnp.float32)
    zeros64 = jnp.zeros((NPAD, D_OUT2), jnp.float32)

    b1r = b1.reshape(1, D_HID)
    g1r = gamma1.reshape(1, D_HID)
    be1r = beta1.reshape(1, D_HID)
    w2p = jnp.pad(W2, ((0, 0), (0, D_OUT2 - D_OUT)))
    b2p = jnp.pad(b2, (0, D_OUT2 - D_OUT)).reshape(1, D_OUT2)

    dega_p, degb_p = _make_deg_kernel()(cold_flat)
    dega = dega_p.reshape(NDEG, 1)
    degb = degb_p.reshape(NDEG, 1)

    hs1a, hs1b, hss1 = _tc_mmprep(x, W1, dega, degb)
    s1a_p, s1b_p = _make_agg1_kernel()(hs1a, hs1b, rowp, colp, zeros128)
    hs2a, hs2b, hss2 = _tc_mid(s1a_p, s1b_p, hss1, dega, degb,
                               b1r, g1r, be1r, w2p)
    s2a_p, s2b_p = _make_agg2_kernel()(hs2a, hs2b, rowd, cold, zeros64)
    return _tc_final(s2a_p, s2b_p, hss2, dega, degb, b2p)


# speedup vs baseline: 12.6086x; 1.0187x over previous
import jax
import jax.numpy as jnp
from jax import lax
from jax.experimental import pallas as pl
from jax.experimental.pallas import tpu as pltpu
from jax.experimental.pallas import tpu_sc as plsc

N_NODES = 10000
N_EDGES = 160000
D_IN = 256
D_HID = 256
D_OUT = 40
D_PAD = 128
D_OUT2 = 48

NS = 16
NC = 2
LB = 32
ND = 8
NBC = 80
EPT = N_EDGES // NS
NB1 = 320
NCH1 = NB1 // NBC
EPW = N_EDGES // (NS * NC)
NB2 = 160
NCH2 = NB2 // NBC
EPWP = 5120
NPAD = 10112
RPT = NPAD // NS
NDEG = 10240
DPT = NDEG // NS

BLK = 2000
GRID = N_NODES // BLK


def _mesh():
    return plsc.VectorSubcoreMesh(core_axis_name="c", subcore_axis_name="s")


def _deg_body(colp, dega, degb, colv, deg_t, redv, outv, stage, sem):
    c = lax.axis_index("c")
    s = lax.axis_index("s")
    wid = s * NC + c
    pltpu.sync_copy(colp.at[wid], colv)
    zero16 = jnp.zeros((16,), jnp.float32)

    def zbody(k, carry):
        deg_t[pl.ds(k * 16, 16)] = zero16
        return carry

    lax.fori_loop(0, NDEG // 16, zbody, 0)

    one16 = jnp.ones((16,), jnp.float32)

    def sbody(k, carry):
        idx = colv[pl.ds(k * 16, 16)]
        plsc.addupdate_scatter(deg_t, [idx], one16)
        return carry

    lax.fori_loop(0, EPWP // 16, sbody, 0)

    pltpu.sync_copy(deg_t, stage.at[s])
    plsc.subcore_barrier()

    pltpu.sync_copy(stage.at[:, pl.ds(s * DPT, DPT)], redv)

    def rbody(j, carry):
        acc = redv[0, pl.ds(j * 16, 16)]
        for t in range(1, NS):
            acc = acc + redv[t, pl.ds(j * 16, 16)]
        outv[pl.ds(j * 16, 16)] = acc
        return carry

    lax.fori_loop(0, DPT // 16, rbody, 0)

    @pl.when(c == 0)
    def _():
        pltpu.sync_copy(outv, dega.at[pl.ds(s * DPT, DPT)])

    @pl.when(c == 1)
    def _():
        pltpu.sync_copy(outv, degb.at[pl.ds(s * DPT, DPT)])


def _make_deg_kernel():
    return pl.kernel(
        _deg_body,
        out_type=(
            jax.ShapeDtypeStruct((NDEG,), jnp.float32),
            jax.ShapeDtypeStruct((NDEG,), jnp.float32),
        ),
        mesh=_mesh(),
        scratch_types=[
            pltpu.VMEM((EPWP,), jnp.int32),
            pltpu.VMEM((NDEG,), jnp.float32),
            pltpu.VMEM((NS, DPT), jnp.float32),
            pltpu.VMEM((DPT,), jnp.float32),
            pltpu.VMEM_SHARED((NS, NDEG), jnp.float32),
            pltpu.SemaphoreType.DMA,
        ],
        compiler_params=pltpu.CompilerParams(needs_layout_passes=False),
    )


def _edge_pipeline(hs, rowv, colv, bufs, acc, sem_g, sem_s):
    nd = len(bufs)
    lead = 6
    lag = nd - lead
    for b in range(lead):
        pltpu.async_copy(hs.at[rowv.at[b]], bufs[b], sem_g)

    def body(q, carry):
        b0 = q * nd
        for j in range(nd):
            b = b0 + j
            jn = (j + lead) % nd
            pltpu.make_async_copy(hs.at[rowv.at[b]], bufs[j], sem_g).wait()
            pltpu.async_copy(bufs[j], acc.at[colv.at[b]], sem_s, add=True)

            @pl.when(b + lead < NBC)
            def _():
                @pl.when(b >= lag)
                def _():
                    pltpu.make_async_copy(
                        bufs[jn], acc.at[colv.at[b - lag]], sem_s).wait()

                pltpu.async_copy(hs.at[rowv.at[b + lead]], bufs[jn], sem_g)

        return carry

    lax.fori_loop(0, NBC // nd, body, 0)
    for j in range(nd):
        pltpu.make_async_copy(bufs[j], acc.at[colv.at[NBC - nd + j]], sem_s).wait()


def _zero_acc(s, zeros, acc):
    pltpu.sync_copy(zeros.at[pl.ds(s * RPT, RPT)], acc.at[pl.ds(s * RPT, RPT)])


def _copy_out(c, s, acc, out_a, out_b):
    @pl.when(c == 0)
    def _():
        pltpu.sync_copy(acc.at[pl.ds(s * RPT, RPT)], out_a.at[pl.ds(s * RPT, RPT)])

    @pl.when(c == 1)
    def _():
        pltpu.sync_copy(acc.at[pl.ds(s * RPT, RPT)], out_b.at[pl.ds(s * RPT, RPT)])


def _agg_scratch(width):
    return ([
        pltpu.VMEM((NBC, LB), jnp.int32),
        pltpu.VMEM((NBC, LB), jnp.int32)]
        + [pltpu.VMEM((LB, width), jnp.float32) for _ in range(ND)]
        + [pltpu.VMEM_SHARED((NPAD, width), jnp.float32),
           pltpu.SemaphoreType.DMA,
           pltpu.SemaphoreType.DMA])


def _agg1_sc_body(hs_a, hs_b, rowp, colp, zeros, out_a, out_b,
                  rowv, colv, *rest):
    bufs = rest[:ND]
    acc, sem_g, sem_s = rest[ND:]
    c = lax.axis_index("c")
    s = lax.axis_index("s")
    _zero_acc(s, zeros, acc)
    plsc.subcore_barrier()

    def run(hs):
        for ci in range(NCH1):
            pltpu.sync_copy(rowp.at[s].at[pl.ds(ci * NBC, NBC)], rowv)
            pltpu.sync_copy(colp.at[s].at[pl.ds(ci * NBC, NBC)], colv)
            _edge_pipeline(hs, rowv, colv, bufs, acc, sem_g, sem_s)

    @pl.when(c == 0)
    def _():
        run(hs_a)

    @pl.when(c == 1)
    def _():
        run(hs_b)

    plsc.subcore_barrier()
    _copy_out(c, s, acc, out_a, out_b)


def _make_agg1_kernel():
    return pl.kernel(
        _agg1_sc_body,
        out_type=(
            jax.ShapeDtypeStruct((NPAD, D_PAD), jnp.float32),
            jax.ShapeDtypeStruct((NPAD, D_PAD), jnp.float32),
        ),
        mesh=_mesh(),
        scratch_types=_agg_scratch(D_PAD),
        compiler_params=pltpu.CompilerParams(use_tc_tiling_on_sc=False),
    )


def _agg2_sc_body(hs_a, hs_b, rowp, colp, zeros, out_a, out_b,
                  rowv, colv, *rest):
    bufs = rest[:ND]
    acc, sem_g, sem_s = rest[ND:]
    c = lax.axis_index("c")
    s = lax.axis_index("s")
    wid = s * NC + c
    _zero_acc(s, zeros, acc)
    plsc.subcore_barrier()

    def run(hs):
        for ci in range(NCH2):
            pltpu.sync_copy(rowp.at[wid].at[pl.ds(ci * NBC, NBC)], rowv)
            pltpu.sync_copy(colp.at[wid].at[pl.ds(ci * NBC, NBC)], colv)
            _edge_pipeline(hs, rowv, colv, bufs, acc, sem_g, sem_s)

    @pl.when(c == 0)
    def _():
        run(hs_a)

    @pl.when(c == 1)
    def _():
        run(hs_b)

    plsc.subcore_barrier()
    _copy_out(c, s, acc, out_a, out_b)


def _make_agg2_kernel():
    return pl.kernel(
        _agg2_sc_body,
        out_type=(
            jax.ShapeDtypeStruct((NPAD, D_OUT2), jnp.float32),
            jax.ShapeDtypeStruct((NPAD, D_OUT2), jnp.float32),
        ),
        mesh=_mesh(),
        scratch_types=_agg_scratch(D_OUT2),
        compiler_params=pltpu.CompilerParams(use_tc_tiling_on_sc=False),
    )


def _dinv_of(dega, degb):
    deg = dega + degb + 1.0
    return lax.rsqrt(deg)


def _mmprep_body(x_ref, w_ref, dega_ref, degb_ref,
                 hs1a_ref, hs1b_ref, hss1_ref):
    h1 = jnp.dot(x_ref[...], w_ref[...], preferred_element_type=jnp.float32)
    dinv = _dinv_of(dega_ref[...], degb_ref[...])
    hs = h1 * dinv
    hs1a_ref[...] = hs[:, :D_PAD]
    hs1b_ref[...] = hs[:, D_PAD:]
    hss1_ref[...] = hs * dinv


def _tc_mmprep(x, w1, dega, degb):
    return pl.pallas_call(
        _mmprep_body,
        grid=(GRID,),
        in_specs=[
            pl.BlockSpec((BLK, D_IN), lambda i: (i, 0)),
            pl.BlockSpec((D_IN, D_HID), lambda i: (0, 0)),
            pl.BlockSpec((BLK, 1), lambda i: (i, 0)),
            pl.BlockSpec((BLK, 1), lambda i: (i, 0)),
        ],
        out_specs=(
            pl.BlockSpec((BLK, D_PAD), lambda i: (i, 0)),
            pl.BlockSpec((BLK, D_PAD), lambda i: (i, 0)),
            pl.BlockSpec((BLK, D_HID), lambda i: (i, 0)),
        ),
        out_shape=(
            jax.ShapeDtypeStruct((N_NODES, D_PAD), jnp.float32),
            jax.ShapeDtypeStruct((N_NODES, D_PAD), jnp.float32),
            jax.ShapeDtypeStruct((N_NODES, D_HID), jnp.float32),
        ),
    )(x, w1, dega, degb)


def _mid_body(s1a_ref, s1b_ref, hss1_ref, dega_ref, degb_ref, b1_ref,
              gamma_ref, beta_ref, w2_ref,
              hs2a_ref, hs2b_ref, hss2_ref, agg_ref, sums_ref, sumsq_ref):
    p = pl.program_id(0)
    j = pl.program_id(1)
    dinv = _dinv_of(dega_ref[...], degb_ref[...])

    @pl.when(p == 0)
    def _():
        s = jnp.concatenate([s1a_ref[...], s1b_ref[...]], axis=1)
        agg = dinv * s + hss1_ref[...] + b1_ref[...]
        agg_ref[pl.ds(j * BLK, BLK), :] = agg

        @pl.when(j == 0)
        def _():
            sums_ref[...] = jnp.zeros_like(sums_ref)
            sumsq_ref[...] = jnp.zeros_like(sumsq_ref)

        sums_ref[...] += jnp.sum(agg, axis=0, keepdims=True)
        sumsq_ref[...] += jnp.sum(agg * agg, axis=0, keepdims=True)

    @pl.when(p == 1)
    def _():
        inv_n = 1.0 / N_NODES
        mu = sums_ref[...] * inv_n
        var = sumsq_ref[...] * inv_n - mu * mu
        scale = gamma_ref[...] * lax.rsqrt(var + 1e-5)
        hn = (agg_ref[pl.ds(j * BLK, BLK), :] - mu) * scale + beta_ref[...]
        hn = jnp.maximum(hn, 0.0)
        h2 = jnp.dot(hn, w2_ref[...], preferred_element_type=jnp.float32)
        hs2 = h2 * dinv
        hs2a_ref[...] = hs2
        hs2b_ref[...] = hs2
        hss2_ref[...] = hs2 * dinv


def _tc_mid(s1a, s1b, hss1, dega, degb, b1, gamma1, beta1, w2p):
    def rowmap(p, j):
        return (j, 0)

    def constmap(p, j):
        return (0, 0)

    def outmap(p, j):
        return (j * p, 0)

    return pl.pallas_call(
        _mid_body,
        grid=(2, GRID),
        in_specs=[
            pl.BlockSpec((BLK, D_PAD), rowmap),
            pl.BlockSpec((BLK, D_PAD), rowmap),
            pl.BlockSpec((BLK, D_HID), rowmap),
            pl.BlockSpec((BLK, 1), rowmap),
            pl.BlockSpec((BLK, 1), rowmap),
            pl.BlockSpec((1, D_HID), constmap),
            pl.BlockSpec((1, D_HID), constmap),
            pl.BlockSpec((1, D_HID), constmap),
            pl.BlockSpec((D_HID, D_OUT2), constmap),
        ],
        out_specs=(
            pl.BlockSpec((BLK, D_OUT2), outmap),
            pl.BlockSpec((BLK, D_OUT2), outmap),
            pl.BlockSpec((BLK, D_OUT2), outmap),
        ),
        out_shape=(
            jax.ShapeDtypeStruct((N_NODES, D_OUT2), jnp.float32),
            jax.ShapeDtypeStruct((N_NODES, D_OUT2), jnp.float32),
            jax.ShapeDtypeStruct((N_NODES, D_OUT2), jnp.float32),
        ),
        scratch_shapes=[
            pltpu.VMEM((N_NODES, D_HID), jnp.float32),
            pltpu.VMEM((1, D_HID), jnp.float32),
            pltpu.VMEM((1, D_HID), jnp.float32),
        ],
    )(s1a, s1b, hss1, dega, degb, b1, gamma1, beta1, w2p)


def _fin_body(s2a_ref, s2b_ref, hss2_ref, dega_ref, degb_ref, b2_ref, o_ref):
    dinv = _dinv_of(dega_ref[...], degb_ref[...])
    s = s2a_ref[...] + s2b_ref[...]
    o_ref[...] = (dinv * s + hss2_ref[...] + b2_ref[...])[:, :D_OUT]


def _tc_final(s2a, s2b, hss2, dega, degb, b2p):
    return pl.pallas_call(
        _fin_body,
        grid=(GRID,),
        in_specs=[
            pl.BlockSpec((BLK, D_OUT2), lambda i: (i, 0)),
            pl.BlockSpec((BLK, D_OUT2), lambda i: (i, 0)),
            pl.BlockSpec((BLK, D_OUT2), lambda i: (i, 0)),
            pl.BlockSpec((BLK, 1), lambda i: (i, 0)),
            pl.BlockSpec((BLK, 1), lambda i: (i, 0)),
            pl.BlockSpec((1, D_OUT2), lambda i: (0, 0)),
        ],
        out_specs=pl.BlockSpec((BLK, D_OUT), lambda i: (i, 0)),
        out_shape=jax.ShapeDtypeStruct((N_NODES, D_OUT), jnp.float32),
    )(s2a, s2b, hss2, dega, degb, b2p)


def kernel(x, edge_index, W1, b1, gamma1, beta1, W2, b2):
    row = edge_index[0].astype(jnp.int32)
    col = edge_index[1].astype(jnp.int32)

    pad1 = NB1 * LB - EPT
    rowp = jnp.pad(row.reshape(NS, EPT), ((0, 0), (0, pad1))).reshape(NS, NB1, LB)
    colp = jnp.pad(col.reshape(NS, EPT), ((0, 0), (0, pad1)),
                   constant_values=N_NODES).reshape(NS, NB1, LB)
    padd = NB2 * LB - EPW
    rowd = jnp.pad(row.reshape(NS * NC, EPW), ((0, 0), (0, padd))
                   ).reshape(NS * NC, NB2, LB)
    cold = jnp.pad(col.reshape(NS * NC, EPW), ((0, 0), (0, padd)),
                   constant_values=N_NODES).reshape(NS * NC, NB2, LB)
    cold_flat = cold.reshape(NS * NC, NB2 * LB)

    zeros128 = jnp.zeros((NPAD, D_PAD), jnp.float32)
    zeros64 = jnp.zeros((NPAD, D_OUT2), jnp.float32)

    b1r = b1.reshape(1, D_HID)
    g1r = gamma1.reshape(1, D_HID)
    be1r = beta1.reshape(1, D_HID)
    w2p = jnp.pad(W2, ((0, 0), (0, D_OUT2 - D_OUT)))
    b2p = jnp.pad(b2, (0, D_OUT2 - D_OUT)).reshape(1, D_OUT2)

    dega_p, degb_p = _make_deg_kernel()(cold_flat)
    dega = dega_p.reshape(NDEG, 1)
    degb = degb_p.reshape(NDEG, 1)

    hs1a, hs1b, hss1 = _tc_mmprep(x, W1, dega, degb)
    s1a_p, s1b_p = _make_agg1_kernel()(hs1a, hs1b, rowp, colp, zeros128)
    hs2a, hs2b, hss2 = _tc_mid(s1a_p, s1b_p, hss1, dega, degb,
                               b1r, g1r, be1r, w2p)
    s2a_p, s2b_p = _make_agg2_kernel()(hs2a, hs2b, rowd, cold, zeros64)
    return _tc_final(s2a_p, s2b_p, hss2, dega, degb, b2p)
